# Initial kernel scaffold; baseline (speedup 1.0000x reference)
import jax
import jax.numpy as jnp
from jax.experimental import pallas as pl


def kernel(x, edge_index, edge_attr, batch, mu, sigma, g, root, b1, W2, b2, W3, b3, Wl, bl):
    raise NotImplementedError("write your pallas kernel here")


# speedup vs baseline: 13.2921x; 13.2921x over previous
import functools

import jax
import jax.numpy as jnp
from jax import lax
from jax.experimental import pallas as pl
from jax.experimental.pallas import tpu as pltpu
from jax.experimental.pallas import tpu_sc as plsc

N = 50000
E = 800000
H = 16
K = 4
D = 3
G = 64
C = 2

NC = 2
NS = 16
NW = NC * NS

NP = 51200
SLAB = NP // NS
EP = 819200
EPW = EP // NW
B = 128
NCHUNK = EPW // B
NSLABCH = SLAB // B

NB = 1024
NGRID = NP // NB

_mesh = plsc.VectorSubcoreMesh(
    core_axis_name="c", subcore_axis_name="s", num_cores=NC, num_subcores=NS
)


def _lane_const(v, dtype=jnp.float32):
    return jnp.full((16,), v, dtype)


@functools.partial(
    pl.kernel,
    out_type=jax.ShapeDtypeStruct((NC, NP, 16), jnp.float32),
    mesh=_mesh,
    compiler_params=pltpu.CompilerParams(needs_layout_passes=False, use_tc_tiling_on_sc=False),
    scratch_types=[
        pltpu.VMEM_SHARED((NP, 16), jnp.float32),
        pltpu.VMEM((B,), jnp.int32),
        pltpu.VMEM((B,), jnp.int32),
        pltpu.VMEM((3 * B,), jnp.float32),
        pltpu.VMEM((B, 8), jnp.float32),
        pltpu.VMEM((B, 16), jnp.float32),
        pltpu.VMEM((384,), jnp.float32),
        pltpu.SemaphoreType.DMA,
    ],
)
def _sc_gmm(eaT, srcp, dstp, x8, coefs, zeros16, out,
            zacc, srcv, dstv, eav, xrows, zbuf, cvm, sem):
    c = lax.axis_index("c")
    s = lax.axis_index("s")
    w = c * NS + s
    r0 = s * SLAB
    pltpu.sync_copy(zeros16.at[pl.ds(0, B)], zbuf)
    for j in range(NSLABCH):
        pltpu.sync_copy(zbuf, zacc.at[pl.ds(r0 + j * B, B)])
    pltpu.sync_copy(coefs, cvm)
    for j in range(B // 16):
        rows = lax.iota(jnp.int32, 16) + (j * 16)
        plsc.store_scatter(zbuf, [rows, _lane_const(12, jnp.int32)],
                           _lane_const(1.0))
        for col in (13, 14, 15):
            plsc.store_scatter(zbuf, [rows, _lane_const(col, jnp.int32)],
                               _lane_const(0.0))
    plsc.subcore_barrier()

    base_w = w * EPW

    def chunk(i, carry):
        base = base_w + i * B
        pltpu.sync_copy(srcp.at[pl.ds(base, B)], srcv)
        pltpu.sync_copy(dstp.at[pl.ds(base, B)], dstv)
        for d in range(3):
            pltpu.sync_copy(eaT.at[pl.ds(d * EP + base, B)],
                            eav.at[pl.ds(d * B, B)])
        pltpu.async_copy(x8.at[srcv], xrows, sem).wait()
        for j in range(B // 16):
            rows = lax.iota(jnp.int32, 16) + (j * 16)
            eas = [eav[pl.ds(d * B + j * 16, 16)] for d in range(3)]
            xc = [plsc.load_gather(xrows, [rows, _lane_const(d, jnp.int32)])
                  for d in range(3)]
            for k in range(K):
                t = None
                for d in range(3):
                    a = cvm[pl.ds((k * 3 + d) * 16, 16)]
                    m = cvm[pl.ds(192 + (k * 3 + d) * 16, 16)]
                    df = eas[d] - m
                    term = a * df * df
                    t = term if t is None else t + term
                gk = jnp.exp(t)
                for d in range(3):
                    plsc.store_scatter(
                        zbuf, [rows, _lane_const(k * 3 + d, jnp.int32)],
                        gk * xc[d])
        pltpu.sync_copy(zbuf, zacc.at[dstv], add=True)
        return carry

    lax.fori_loop(0, NCHUNK, chunk, 0)
    plsc.subcore_barrier()
    for j in range(NSLABCH):
        pltpu.sync_copy(zacc.at[pl.ds(r0 + j * B, B)], zbuf)
        pltpu.sync_copy(zbuf, out.at[c, pl.ds(r0 + j * B, B)])


@functools.partial(
    pl.kernel,
    out_type=jax.ShapeDtypeStruct((NC, NP, 16), jnp.float32),
    mesh=_mesh,
    compiler_params=pltpu.CompilerParams(needs_layout_passes=False, use_tc_tiling_on_sc=False),
    scratch_types=[
        pltpu.VMEM_SHARED((NP, 16), jnp.float32),
        pltpu.VMEM((B,), jnp.int32),
        pltpu.VMEM((B,), jnp.int32),
        pltpu.VMEM((B, 16), jnp.float32),
        pltpu.SemaphoreType.DMA,
    ],
)
def _sc_gcn(u16, srcp, dstp, zeros16, out, acc, srcv, dstv, rows_v, sem):
    c = lax.axis_index("c")
    s = lax.axis_index("s")
    w = c * NS + s
    r0 = s * SLAB
    pltpu.sync_copy(zeros16.at[pl.ds(0, B)], rows_v)
    for j in range(NSLABCH):
        pltpu.sync_copy(rows_v, acc.at[pl.ds(r0 + j * B, B)])
    plsc.subcore_barrier()

    base_w = w * EPW

    def chunk(i, carry):
        base = base_w + i * B
        pltpu.sync_copy(srcp.at[pl.ds(base, B)], srcv)
        pltpu.sync_copy(dstp.at[pl.ds(base, B)], dstv)
        pltpu.async_copy(u16.at[srcv], rows_v, sem).wait()
        pltpu.sync_copy(rows_v, acc.at[dstv], add=True)
        return carry

    lax.fori_loop(0, NCHUNK, chunk, 0)
    plsc.subcore_barrier()
    for j in range(NSLABCH):
        pltpu.sync_copy(acc.at[pl.ds(r0 + j * B, B)], rows_v)
        pltpu.sync_copy(rows_v, out.at[c, pl.ds(r0 + j * B, B)])


def _tc_a_body(zp, x8b, g2, r8, b1, u2o, dvo):
    Z2 = zp[0] + zp[1]
    m12 = lax.broadcasted_iota(jnp.int32, (1, 16), 1) == 12
    cnt = jnp.sum(jnp.where(m12, Z2, 0.0), axis=1, keepdims=True)
    rcp = 1.0 / jnp.maximum(cnt, 1.0)
    agg = jnp.dot(Z2, g2[...], preferred_element_type=jnp.float32) * rcp
    xr = jnp.dot(x8b[...], r8[...], preferred_element_type=jnp.float32)
    h1 = jnp.maximum(agg + xr + b1[0:1, :], 0.0)
    dinv = lax.rsqrt(cnt + 1.0)
    gidx = lax.broadcasted_iota(jnp.int32, (NB, 1), 0) + pl.program_id(0) * NB
    mask = gidx < N
    u2o[...] = jnp.where(mask, h1 * dinv, 0.0)
    dvo[...] = jnp.broadcast_to(jnp.where(mask, dinv, 1.0), (NB, 16))


def _tc_a(zpair, x8, g2p, root8, b1p):
    return pl.pallas_call(
        _tc_a_body,
        grid=(NGRID,),
        in_specs=[
            pl.BlockSpec((NC, NB, 16), lambda i: (0, i, 0)),
            pl.BlockSpec((NB, 8), lambda i: (i, 0)),
            pl.BlockSpec((16, 16), lambda i: (0, 0)),
            pl.BlockSpec((8, 16), lambda i: (0, 0)),
            pl.BlockSpec((8, 16), lambda i: (0, 0)),
        ],
        out_specs=[
            pl.BlockSpec((NB, 16), lambda i: (i, 0)),
            pl.BlockSpec((NB, 16), lambda i: (i, 0)),
        ],
        out_shape=[
            jax.ShapeDtypeStruct((NP, 16), jnp.float32),
            jax.ShapeDtypeStruct((NP, 16), jnp.float32),
        ],
    )(zpair, x8, g2p, root8, b1p)


def _tc_b_body(sp, ub, dv, w, b, uo, *, relu):
    P = dv[...] * (sp[0] + sp[1] + ub[...])
    h = jnp.dot(P, w[...], preferred_element_type=jnp.float32) + b[0:1, :]
    if relu:
        h = jnp.maximum(h, 0.0)
    gidx = lax.broadcasted_iota(jnp.int32, (NB, 1), 0) + pl.program_id(0) * NB
    uo[...] = jnp.where(gidx < N, h * dv[...], 0.0)


def _tc_b(spair, u, dv16, wp, bp, relu):
    return pl.pallas_call(
        functools.partial(_tc_b_body, relu=relu),
        grid=(NGRID,),
        in_specs=[
            pl.BlockSpec((NC, NB, 16), lambda i: (0, i, 0)),
            pl.BlockSpec((NB, 16), lambda i: (i, 0)),
            pl.BlockSpec((NB, 16), lambda i: (i, 0)),
            pl.BlockSpec((16, 16), lambda i: (0, 0)),
            pl.BlockSpec((8, 16), lambda i: (0, 0)),
        ],
        out_specs=pl.BlockSpec((NB, 16), lambda i: (i, 0)),
        out_shape=jax.ShapeDtypeStruct((NP, 16), jnp.float32),
    )(spair, u, dv16, wp, bp)


def _tc_c_body(sp, ub, dv, w3, b3, bb, wl, bl, outo, acc):
    i = pl.program_id(0)

    @pl.when(i == 0)
    def _init():
        acc[...] = jnp.zeros((G, 32), jnp.float32)

    P = dv[...] * (sp[0] + sp[1] + ub[...])
    h3 = jnp.dot(P, w3[...], preferred_element_type=jnp.float32) + b3[0:1, :]
    oh = (bb[...] == lax.broadcasted_iota(jnp.int32, (1, G), 1)).astype(jnp.float32)
    acc[:, 0:16] += lax.dot_general(
        oh, h3, (((0,), (0,)), ((), ())), preferred_element_type=jnp.float32)
    acc[:, 16:17] += jnp.sum(oh, axis=0)[:, None]

    @pl.when(i == NGRID - 1)
    def _fin():
        gc = jnp.maximum(acc[:, 16:17], 1.0)
        pooled = acc[:, 0:16] / gc
        logits = jnp.dot(pooled, wl[...], preferred_element_type=jnp.float32) + bl[0:1, :]
        colmask = lax.broadcasted_iota(jnp.int32, (1, 8), 1) < C
        lm = jnp.where(colmask, logits, -1e30)
        mx = jnp.max(lm, axis=1, keepdims=True)
        ex = jnp.where(colmask, jnp.exp(lm - mx), 0.0)
        outo[...] = ex / jnp.sum(ex, axis=1, keepdims=True)


def _tc_c(spair, u3, dv16, w3p, b3p, batchcol, wlp, blp):
    return pl.pallas_call(
        _tc_c_body,
        grid=(NGRID,),
        in_specs=[
            pl.BlockSpec((NC, NB, 16), lambda i: (0, i, 0)),
            pl.BlockSpec((NB, 16), lambda i: (i, 0)),
            pl.BlockSpec((NB, 16), lambda i: (i, 0)),
            pl.BlockSpec((16, 16), lambda i: (0, 0)),
            pl.BlockSpec((8, 16), lambda i: (0, 0)),
            pl.BlockSpec((NB, 1), lambda i: (i, 0)),
            pl.BlockSpec((16, 8), lambda i: (0, 0)),
            pl.BlockSpec((8, 8), lambda i: (0, 0)),
        ],
        out_specs=pl.BlockSpec((G, 8), lambda i: (0, 0)),
        out_shape=jax.ShapeDtypeStruct((G, 8), jnp.float32),
        scratch_shapes=[pltpu.VMEM((G, 32), jnp.float32)],
    )(spair, u3, dv16, w3p, b3p, batchcol, wlp, blp)


def kernel(x, edge_index, edge_attr, batch, mu, sigma, g, root,
           b1, W2, b2, W3, b3, Wl, bl):
    f32 = jnp.float32
    src = edge_index[0]
    dst = edge_index[1]

    pad_e = EP - E
    srcp = jnp.concatenate([src, jnp.full((pad_e,), NP - 1, jnp.int32)])
    dstp = jnp.concatenate([dst, jnp.full((pad_e,), NP - 1, jnp.int32)])
    eaT = jnp.concatenate(
        [edge_attr.T, jnp.zeros((D, pad_e), f32)], axis=1).reshape(-1)
    x8 = jnp.zeros((NP, 8), f32).at[:N, :D].set(x)
    zeros16 = jnp.zeros((NP, 16), f32)

    a = -0.5 / (1e-15 + sigma * sigma)
    coefs = jnp.concatenate([
        jnp.repeat(a.reshape(-1), 16),
        jnp.repeat(mu.reshape(-1), 16),
    ]).astype(f32)

    g2p = jnp.zeros((16, 16), f32).at[:K * D, :].set(
        g.reshape(D, K, H).transpose(1, 0, 2).reshape(K * D, H))
    root8 = jnp.zeros((8, 16), f32).at[:D, :].set(root)
    b1p = jnp.broadcast_to(b1, (8, 16)).astype(f32)
    w2p = W2.astype(f32)
    b2p = jnp.broadcast_to(b2, (8, 16)).astype(f32)
    w3p = W3.astype(f32)
    b3p = jnp.broadcast_to(b3, (8, 16)).astype(f32)
    wlp = jnp.zeros((16, 8), f32).at[:, :C].set(Wl)
    blp = jnp.zeros((8, 8), f32).at[:, :C].set(jnp.broadcast_to(bl, (8, C)))
    batchcol = jnp.concatenate(
        [batch, jnp.full((NP - N,), G, jnp.int32)]).reshape(NP, 1)

    zpair = _sc_gmm(eaT, srcp, dstp, x8, coefs, zeros16)
    u2, dv16 = _tc_a(zpair, x8, g2p, root8, b1p)
    s2 = _sc_gcn(u2, srcp, dstp, zeros16)
    u3 = _tc_b(s2, u2, dv16, w2p, b2p, relu=True)
    s3 = _sc_gcn(u3, srcp, dstp, zeros16)
    out8 = _tc_c(s3, u3, dv16, w3p, b3p, batchcol, wlp, blp)
    return out8[:, :C]


# speedup vs baseline: 17.8502x; 1.3429x over previous
import functools

import jax
import jax.numpy as jnp
from jax import lax
from jax.experimental import pallas as pl
from jax.experimental.pallas import tpu as pltpu
from jax.experimental.pallas import tpu_sc as plsc

N = 50000
E = 800000
H = 16
K = 4
D = 3
G = 64
C = 2

NC = 2
NS = 16
NW = NC * NS

NP = 51200
SLAB = NP // NS
EP = 819200
EPW = EP // NW
B = 128
NCHUNK = EPW // B
NSLABCH = SLAB // B

NB = 1024
NGRID = NP // NB

_mesh = plsc.VectorSubcoreMesh(
    core_axis_name="c", subcore_axis_name="s", num_cores=NC, num_subcores=NS
)


def _lane_const(v, dtype=jnp.float32):
    return jnp.full((16,), v, dtype)


@functools.partial(
    pl.kernel,
    out_type=jax.ShapeDtypeStruct((NC, NP, 16), jnp.float32),
    mesh=_mesh,
    compiler_params=pltpu.CompilerParams(needs_layout_passes=False, use_tc_tiling_on_sc=False),
    scratch_types=[
        pltpu.VMEM_SHARED((NP, 16), jnp.float32),
        pltpu.VMEM((B,), jnp.int32),
        pltpu.VMEM((B,), jnp.int32),
        pltpu.VMEM((3 * B,), jnp.float32),
        pltpu.VMEM((B, 8), jnp.float32),
        pltpu.VMEM((B, 16), jnp.float32),
        pltpu.VMEM((384,), jnp.float32),
        pltpu.SemaphoreType.DMA,
    ],
)
def _sc_gmm(eaT, srcp, dstp, x8, coefs, zeros16, out,
            zacc, srcv, dstv, eav, xrows, zbuf, cvm, sem):
    c = lax.axis_index("c")
    s = lax.axis_index("s")
    w = c * NS + s
    r0 = s * SLAB
    pltpu.sync_copy(zeros16.at[pl.ds(0, B)], zbuf)
    for j in range(NSLABCH):
        pltpu.sync_copy(zbuf, zacc.at[pl.ds(r0 + j * B, B)])
    pltpu.sync_copy(coefs, cvm)
    for j in range(B // 16):
        rows = lax.iota(jnp.int32, 16) + (j * 16)
        plsc.store_scatter(zbuf, [rows, _lane_const(12, jnp.int32)],
                           _lane_const(1.0))
        for col in (13, 14, 15):
            plsc.store_scatter(zbuf, [rows, _lane_const(col, jnp.int32)],
                               _lane_const(0.0))
    plsc.subcore_barrier()

    base_w = w * EPW

    def chunk(i, carry):
        base = base_w + i * B
        pltpu.sync_copy(srcp.at[pl.ds(base, B)], srcv)
        pltpu.sync_copy(dstp.at[pl.ds(base, B)], dstv)
        for d in range(3):
            pltpu.sync_copy(eaT.at[pl.ds(d * EP + base, B)],
                            eav.at[pl.ds(d * B, B)])
        pltpu.async_copy(x8.at[srcv], xrows, sem).wait()
        for j in range(B // 16):
            rows = lax.iota(jnp.int32, 16) + (j * 16)
            eas = [eav[pl.ds(d * B + j * 16, 16)] for d in range(3)]
            xc = [plsc.load_gather(xrows, [rows, _lane_const(d, jnp.int32)])
                  for d in range(3)]
            for k in range(K):
                t = None
                for d in range(3):
                    a = cvm[pl.ds((k * 3 + d) * 16, 16)]
                    m = cvm[pl.ds(192 + (k * 3 + d) * 16, 16)]
                    df = eas[d] - m
                    term = a * df * df
                    t = term if t is None else t + term
                gk = jnp.exp(t)
                for d in range(3):
                    plsc.store_scatter(
                        zbuf, [rows, _lane_const(k * 3 + d, jnp.int32)],
                        gk * xc[d])
        pltpu.sync_copy(zbuf, zacc.at[dstv], add=True)
        return carry

    lax.fori_loop(0, NCHUNK, chunk, 0)
    plsc.subcore_barrier()
    for j in range(NSLABCH):
        pltpu.sync_copy(zacc.at[pl.ds(r0 + j * B, B)], zbuf)
        pltpu.sync_copy(zbuf, out.at[c, pl.ds(r0 + j * B, B)])


@functools.partial(
    pl.kernel,
    out_type=jax.ShapeDtypeStruct((NC, NP, 16), jnp.float32),
    mesh=_mesh,
    compiler_params=pltpu.CompilerParams(needs_layout_passes=False, use_tc_tiling_on_sc=False),
    scratch_types=[
        pltpu.VMEM_SHARED((NP, 16), jnp.float32),
        pltpu.VMEM((8, B), jnp.int32),
        pltpu.VMEM((8, B), jnp.int32),
        pltpu.VMEM((4, B, 16), jnp.float32),
        pltpu.VMEM((B, 16), jnp.float32),
        pltpu.SemaphoreType.DMA((8,)),
        pltpu.SemaphoreType.DMA((4,)),
        pltpu.SemaphoreType.DMA((4,)),
        pltpu.SemaphoreType.DMA((2,)),
        pltpu.SemaphoreType.DMA,
    ],
)
def _sc_gcn(u16, srcp, dstp, zeros16, out,
            acc, srcv2, dstv2, rows2, zb, isem, gsem, ssem, osem, zsem):
    c = lax.axis_index("c")
    s = lax.axis_index("s")
    w = c * NS + s
    r0 = s * SLAB
    base_w = w * EPW

    pltpu.sync_copy(zeros16.at[pl.ds(0, B)], zb)
    for j in range(NSLABCH):
        pltpu.async_copy(zb, acc.at[pl.ds(r0 + j * B, B)], zsem)
    for j in range(NSLABCH):
        pltpu.make_async_copy(zb, acc.at[pl.ds(r0, B)], zsem).wait()
    plsc.subcore_barrier()

    def s_idx(i, b):
        pltpu.async_copy(srcp.at[pl.ds(base_w + i * B, B)], srcv2.at[b],
                         isem.at[b])
        pltpu.async_copy(dstp.at[pl.ds(base_w + i * B, B)], dstv2.at[b],
                         isem.at[b])

    def w_idx(b):
        pltpu.make_async_copy(srcp.at[pl.ds(0, B)], srcv2.at[b],
                              isem.at[b]).wait()
        pltpu.make_async_copy(dstp.at[pl.ds(0, B)], dstv2.at[b],
                              isem.at[b]).wait()

    def s_gather(b, b4):
        pltpu.async_copy(u16.at[srcv2.at[b]], rows2.at[b4], gsem.at[b4])

    def w_gather(b4):
        pltpu.make_async_copy(u16.at[pl.ds(0, B)], rows2.at[b4],
                              gsem.at[b4]).wait()

    def s_scatter(b, b4):
        pltpu.async_copy(rows2.at[b4], acc.at[dstv2.at[b]], ssem.at[b4],
                         add=True)

    def w_scatter(b4):
        pltpu.make_async_copy(u16.at[pl.ds(0, B)], rows2.at[b4],
                              ssem.at[b4]).wait()

    for i in range(3):
        s_idx(i, i)
    for i in range(8):
        b4 = i % 4
        if i >= 4:
            w_scatter(b4)
        w_idx(i)
        s_gather(i, b4)
        if i >= 1:
            w_gather((i - 1) % 4)
            s_scatter(i - 1, (i - 1) % 4)
        s_idx(i + 3, (i + 3) % 8)

    def outer(o, carry):
        for b in range(8):
            i = o * 8 + b
            b4 = b % 4
            w_scatter(b4)
            w_idx(b)
            s_gather(b, b4)
            pb, pb4 = (b - 1) % 8, (b - 1) % 4
            w_gather(pb4)
            s_scatter(pb, pb4)

            @pl.when(i + 3 < NCHUNK)
            def _():
                s_idx(i + 3, (b + 3) % 8)
        return carry

    lax.fori_loop(1, NCHUNK // 8, outer, 0)

    w_gather(3)
    s_scatter(7, 3)
    for b4 in range(4):
        w_scatter(b4)
    plsc.subcore_barrier()

    for j in range(NSLABCH):
        bj = j % 2
        if j >= 2:
            pltpu.make_async_copy(u16.at[pl.ds(0, B)], zb, osem.at[bj]).wait()
        pltpu.sync_copy(acc.at[pl.ds(r0 + j * B, B)], rows2.at[bj])
        pltpu.async_copy(rows2.at[bj], out.at[c, pl.ds(r0 + j * B, B)],
                         osem.at[bj])
    for bj in ((NSLABCH - 2) % 2, (NSLABCH - 1) % 2):
        pltpu.make_async_copy(u16.at[pl.ds(0, B)], zb, osem.at[bj]).wait()


def _tc_a_body(zp, x8b, g2, r8, b1, u2o, dvo):
    Z2 = zp[0] + zp[1]
    m12 = lax.broadcasted_iota(jnp.int32, (1, 16), 1) == 12
    cnt = jnp.sum(jnp.where(m12, Z2, 0.0), axis=1, keepdims=True)
    rcp = 1.0 / jnp.maximum(cnt, 1.0)
    agg = jnp.dot(Z2, g2[...], preferred_element_type=jnp.float32) * rcp
    xr = jnp.dot(x8b[...], r8[...], preferred_element_type=jnp.float32)
    h1 = jnp.maximum(agg + xr + b1[0:1, :], 0.0)
    dinv = lax.rsqrt(cnt + 1.0)
    gidx = lax.broadcasted_iota(jnp.int32, (NB, 1), 0) + pl.program_id(0) * NB
    mask = gidx < N
    u2o[...] = jnp.where(mask, h1 * dinv, 0.0)
    dvo[...] = jnp.broadcast_to(jnp.where(mask, dinv, 1.0), (NB, 16))


def _tc_a(zpair, x8, g2p, root8, b1p):
    return pl.pallas_call(
        _tc_a_body,
        grid=(NGRID,),
        in_specs=[
            pl.BlockSpec((NC, NB, 16), lambda i: (0, i, 0)),
            pl.BlockSpec((NB, 8), lambda i: (i, 0)),
            pl.BlockSpec((16, 16), lambda i: (0, 0)),
            pl.BlockSpec((8, 16), lambda i: (0, 0)),
            pl.BlockSpec((8, 16), lambda i: (0, 0)),
        ],
        out_specs=[
            pl.BlockSpec((NB, 16), lambda i: (i, 0)),
            pl.BlockSpec((NB, 16), lambda i: (i, 0)),
        ],
        out_shape=[
            jax.ShapeDtypeStruct((NP, 16), jnp.float32),
            jax.ShapeDtypeStruct((NP, 16), jnp.float32),
        ],
    )(zpair, x8, g2p, root8, b1p)


def _tc_b_body(sp, ub, dv, w, b, uo, *, relu):
    P = dv[...] * (sp[0] + sp[1] + ub[...])
    h = jnp.dot(P, w[...], preferred_element_type=jnp.float32) + b[0:1, :]
    if relu:
        h = jnp.maximum(h, 0.0)
    gidx = lax.broadcasted_iota(jnp.int32, (NB, 1), 0) + pl.program_id(0) * NB
    uo[...] = jnp.where(gidx < N, h * dv[...], 0.0)


def _tc_b(spair, u, dv16, wp, bp, relu):
    return pl.pallas_call(
        functools.partial(_tc_b_body, relu=relu),
        grid=(NGRID,),
        in_specs=[
            pl.BlockSpec((NC, NB, 16), lambda i: (0, i, 0)),
            pl.BlockSpec((NB, 16), lambda i: (i, 0)),
            pl.BlockSpec((NB, 16), lambda i: (i, 0)),
            pl.BlockSpec((16, 16), lambda i: (0, 0)),
            pl.BlockSpec((8, 16), lambda i: (0, 0)),
        ],
        out_specs=pl.BlockSpec((NB, 16), lambda i: (i, 0)),
        out_shape=jax.ShapeDtypeStruct((NP, 16), jnp.float32),
    )(spair, u, dv16, wp, bp)


def _tc_c_body(sp, ub, dv, w3, b3, bb, wl, bl, outo, acc):
    i = pl.program_id(0)

    @pl.when(i == 0)
    def _init():
        acc[...] = jnp.zeros((G, 32), jnp.float32)

    P = dv[...] * (sp[0] + sp[1] + ub[...])
    h3 = jnp.dot(P, w3[...], preferred_element_type=jnp.float32) + b3[0:1, :]
    oh = (bb[...] == lax.broadcasted_iota(jnp.int32, (1, G), 1)).astype(jnp.float32)
    acc[:, 0:16] += lax.dot_general(
        oh, h3, (((0,), (0,)), ((), ())), preferred_element_type=jnp.float32)
    acc[:, 16:17] += jnp.sum(oh, axis=0)[:, None]

    @pl.when(i == NGRID - 1)
    def _fin():
        gc = jnp.maximum(acc[:, 16:17], 1.0)
        pooled = acc[:, 0:16] / gc
        logits = jnp.dot(pooled, wl[...], preferred_element_type=jnp.float32) + bl[0:1, :]
        colmask = lax.broadcasted_iota(jnp.int32, (1, 8), 1) < C
        lm = jnp.where(colmask, logits, -1e30)
        mx = jnp.max(lm, axis=1, keepdims=True)
        ex = jnp.where(colmask, jnp.exp(lm - mx), 0.0)
        outo[...] = ex / jnp.sum(ex, axis=1, keepdims=True)


def _tc_c(spair, u3, dv16, w3p, b3p, batchcol, wlp, blp):
    return pl.pallas_call(
        _tc_c_body,
        grid=(NGRID,),
        in_specs=[
            pl.BlockSpec((NC, NB, 16), lambda i: (0, i, 0)),
            pl.BlockSpec((NB, 16), lambda i: (i, 0)),
            pl.BlockSpec((NB, 16), lambda i: (i, 0)),
            pl.BlockSpec((16, 16), lambda i: (0, 0)),
            pl.BlockSpec((8, 16), lambda i: (0, 0)),
            pl.BlockSpec((NB, 1), lambda i: (i, 0)),
            pl.BlockSpec((16, 8), lambda i: (0, 0)),
            pl.BlockSpec((8, 8), lambda i: (0, 0)),
        ],
        out_specs=pl.BlockSpec((G, 8), lambda i: (0, 0)),
        out_shape=jax.ShapeDtypeStruct((G, 8), jnp.float32),
        scratch_shapes=[pltpu.VMEM((G, 32), jnp.float32)],
    )(spair, u3, dv16, w3p, b3p, batchcol, wlp, blp)


def kernel(x, edge_index, edge_attr, batch, mu, sigma, g, root,
           b1, W2, b2, W3, b3, Wl, bl):
    f32 = jnp.float32
    src = edge_index[0]
    dst = edge_index[1]

    pad_e = EP - E
    srcp = jnp.concatenate([src, jnp.full((pad_e,), NP - 1, jnp.int32)])
    dstp = jnp.concatenate([dst, jnp.full((pad_e,), NP - 1, jnp.int32)])
    eaT = jnp.concatenate(
        [edge_attr.T, jnp.zeros((D, pad_e), f32)], axis=1).reshape(-1)
    x8 = jnp.zeros((NP, 8), f32).at[:N, :D].set(x)
    zeros16 = jnp.zeros((NP, 16), f32)

    a = -0.5 / (1e-15 + sigma * sigma)
    coefs = jnp.concatenate([
        jnp.repeat(a.reshape(-1), 16),
        jnp.repeat(mu.reshape(-1), 16),
    ]).astype(f32)

    g2p = jnp.zeros((16, 16), f32).at[:K * D, :].set(
        g.reshape(D, K, H).transpose(1, 0, 2).reshape(K * D, H))
    root8 = jnp.zeros((8, 16), f32).at[:D, :].set(root)
    b1p = jnp.broadcast_to(b1, (8, 16)).astype(f32)
    w2p = W2.astype(f32)
    b2p = jnp.broadcast_to(b2, (8, 16)).astype(f32)
    w3p = W3.astype(f32)
    b3p = jnp.broadcast_to(b3, (8, 16)).astype(f32)
    wlp = jnp.zeros((16, 8), f32).at[:, :C].set(Wl)
    blp = jnp.zeros((8, 8), f32).at[:, :C].set(jnp.broadcast_to(bl, (8, C)))
    batchcol = jnp.concatenate(
        [batch, jnp.full((NP - N,), G, jnp.int32)]).reshape(NP, 1)

    zpair = _sc_gmm(eaT, srcp, dstp, x8, coefs, zeros16)
    u2, dv16 = _tc_a(zpair, x8, g2p, root8, b1p)
    s2 = _sc_gcn(u2, srcp, dstp, zeros16)
    u3 = _tc_b(s2, u2, dv16, w2p, b2p, relu=True)
    s3 = _sc_gcn(u3, srcp, dstp, zeros16)
    out8 = _tc_c(s3, u3, dv16, w3p, b3p, batchcol, wlp, blp)
    return out8[:, :C]


# speedup vs baseline: 28.8155x; 1.6143x over previous
import functools

import jax
import jax.numpy as jnp
from jax import lax
from jax.experimental import pallas as pl
from jax.experimental.pallas import tpu as pltpu
from jax.experimental.pallas import tpu_sc as plsc

N = 50000
E = 800000
H = 16
K = 4
D = 3
G = 64
C = 2

NC = 2
NS = 16
NW = NC * NS

NP = 51200
SLAB = NP // NS
EP = 819200
EPW = EP // NW
B = 128
NCHUNK = EPW // B
NSLABCH = SLAB // B

NB = 1024
NGRID = NP // NB

_mesh = plsc.VectorSubcoreMesh(
    core_axis_name="c", subcore_axis_name="s", num_cores=NC, num_subcores=NS
)


def _lane_const(v, dtype=jnp.float32):
    return jnp.full((16,), v, dtype)


@functools.partial(
    pl.kernel,
    out_type=jax.ShapeDtypeStruct((NC, NP, 16), jnp.float32),
    mesh=_mesh,
    compiler_params=pltpu.CompilerParams(needs_layout_passes=False, use_tc_tiling_on_sc=False),
    scratch_types=[
        pltpu.VMEM_SHARED((NP, 16), jnp.float32),
        pltpu.VMEM((8, B), jnp.int32),
        pltpu.VMEM((8, B), jnp.int32),
        pltpu.VMEM((8 * 3 * B,), jnp.float32),
        pltpu.VMEM((2, B, 8), jnp.float32),
        pltpu.VMEM((2, B, 16), jnp.float32),
        pltpu.VMEM((384,), jnp.float32),
        pltpu.SemaphoreType.DMA((8,)),
        pltpu.SemaphoreType.DMA((2,)),
        pltpu.SemaphoreType.DMA((2,)),
        pltpu.SemaphoreType.DMA((2,)),
        pltpu.SemaphoreType.DMA,
    ],
)
def _sc_gmm(eaT, srcp, dstp, x8, coefs, zeros16, out,
            zacc, srcv2, dstv2, eav2, xrows2, zbuf2, cvm,
            isem, gsem, ssem, osem, zsem):
    c = lax.axis_index("c")
    s = lax.axis_index("s")
    w = c * NS + s
    r0 = s * SLAB
    base_w = w * EPW

    pltpu.sync_copy(zeros16.at[pl.ds(0, B)], zbuf2.at[0])
    for j in range(NSLABCH):
        pltpu.async_copy(zbuf2.at[0], zacc.at[pl.ds(r0 + j * B, B)], zsem)
    pltpu.sync_copy(coefs, cvm)
    for j in range(NSLABCH):
        pltpu.make_async_copy(zbuf2.at[0], zacc.at[pl.ds(r0, B)], zsem).wait()
    for z2 in range(2):
        for j in range(B // 16):
            rows = lax.iota(jnp.int32, 16) + (j * 16)
            plsc.store_scatter(
                zbuf2, [_lane_const(z2, jnp.int32), rows,
                        _lane_const(12, jnp.int32)], _lane_const(1.0))
            for col in (13, 14, 15):
                plsc.store_scatter(
                    zbuf2, [_lane_const(z2, jnp.int32), rows,
                            _lane_const(col, jnp.int32)], _lane_const(0.0))
    plsc.subcore_barrier()

    def s_idx(i, b):
        base = base_w + i * B
        pltpu.async_copy(srcp.at[pl.ds(base, B)], srcv2.at[b], isem.at[b])
        pltpu.async_copy(dstp.at[pl.ds(base, B)], dstv2.at[b], isem.at[b])
        for d in range(3):
            pltpu.async_copy(eaT.at[pl.ds(d * EP + base, B)],
                             eav2.at[pl.ds((b * 3 + d) * B, B)], isem.at[b])

    def w_idx(b):
        pltpu.make_async_copy(srcp.at[pl.ds(0, B)], srcv2.at[b],
                              isem.at[b]).wait()
        pltpu.make_async_copy(dstp.at[pl.ds(0, B)], dstv2.at[b],
                              isem.at[b]).wait()
        for d in range(3):
            pltpu.make_async_copy(eaT.at[pl.ds(0, B)],
                                  eav2.at[pl.ds((b * 3 + d) * B, B)],
                                  isem.at[b]).wait()

    def s_gather(i, b, g2):
        pltpu.async_copy(x8.at[srcv2.at[b]], xrows2.at[g2], gsem.at[g2])

    def w_gather(g2):
        pltpu.make_async_copy(x8.at[pl.ds(0, B)], xrows2.at[g2],
                              gsem.at[g2]).wait()

    def s_scatter(b, z2):
        pltpu.async_copy(zbuf2.at[z2], zacc.at[dstv2.at[b]], ssem.at[z2],
                         add=True)

    def w_scatter(z2):
        pltpu.make_async_copy(zeros16.at[pl.ds(0, B)], zbuf2.at[z2],
                              ssem.at[z2]).wait()

    av = [cvm[pl.ds(t * 16, 16)] for t in range(12)]
    mv = [cvm[pl.ds(192 + t * 16, 16)] for t in range(12)]

    def compute(b, g2, z2):
        for j in range(B // 16):
            rows = lax.iota(jnp.int32, 16) + (j * 16)
            eas = [eav2[pl.ds((b * 3 + d) * B + j * 16, 16)] for d in range(3)]
            xc = [plsc.load_gather(
                      xrows2, [_lane_const(g2, jnp.int32), rows,
                               _lane_const(d, jnp.int32)])
                  for d in range(3)]
            zsel = _lane_const(z2, jnp.int32)
            for k in range(K):
                t = None
                for d in range(3):
                    df = eas[d] - mv[k * 3 + d]
                    term = av[k * 3 + d] * df * df
                    t = term if t is None else t + term
                gk = jnp.exp(t)
                for d in range(3):
                    plsc.store_scatter(
                        zbuf2, [zsel, rows, _lane_const(k * 3 + d, jnp.int32)],
                        gk * xc[d])

    for i in range(4):
        s_idx(i, i)
    w_idx(0)
    s_gather(0, 0, 0)

    def outer(o, carry):
        for b in range(8):
            i = o * 8 + b

            @pl.when(i + 1 < NCHUNK)
            def _():
                w_idx((b + 1) % 8)
                s_gather(i + 1, (b + 1) % 8, (b + 1) % 2)

            @pl.when(i >= 2)
            def _():
                w_scatter(b % 2)

            w_gather(b % 2)
            compute(b, b % 2, b % 2)
            s_scatter(b, b % 2)

            @pl.when(i + 4 < NCHUNK)
            def _():
                s_idx(i + 4, (b + 4) % 8)
        return carry

    lax.fori_loop(0, NCHUNK // 8, outer, 0)

    for z2 in range(2):
        w_scatter(z2)
    plsc.subcore_barrier()

    for j in range(NSLABCH):
        bj = j % 2
        if j >= 2:
            pltpu.make_async_copy(zeros16.at[pl.ds(0, B)], zbuf2.at[bj],
                                  osem.at[bj]).wait()
        pltpu.sync_copy(zacc.at[pl.ds(r0 + j * B, B)], zbuf2.at[bj])
        pltpu.async_copy(zbuf2.at[bj], out.at[c, pl.ds(r0 + j * B, B)],
                         osem.at[bj])
    for bj in ((NSLABCH - 2) % 2, (NSLABCH - 1) % 2):
        pltpu.make_async_copy(zeros16.at[pl.ds(0, B)], zbuf2.at[bj],
                              osem.at[bj]).wait()


@functools.partial(
    pl.kernel,
    out_type=jax.ShapeDtypeStruct((NC, NP, 16), jnp.float32),
    mesh=_mesh,
    compiler_params=pltpu.CompilerParams(needs_layout_passes=False, use_tc_tiling_on_sc=False),
    scratch_types=[
        pltpu.VMEM_SHARED((NP, 16), jnp.float32),
        pltpu.VMEM((8, B), jnp.int32),
        pltpu.VMEM((8, B), jnp.int32),
        pltpu.VMEM((4, B, 16), jnp.float32),
        pltpu.VMEM((B, 16), jnp.float32),
        pltpu.SemaphoreType.DMA((8,)),
        pltpu.SemaphoreType.DMA((4,)),
        pltpu.SemaphoreType.DMA((4,)),
        pltpu.SemaphoreType.DMA((2,)),
        pltpu.SemaphoreType.DMA,
    ],
)
def _sc_gcn(u16, srcp, dstp, zeros16, out,
            acc, srcv2, dstv2, rows2, zb, isem, gsem, ssem, osem, zsem):
    c = lax.axis_index("c")
    s = lax.axis_index("s")
    w = c * NS + s
    r0 = s * SLAB
    base_w = w * EPW

    pltpu.sync_copy(zeros16.at[pl.ds(0, B)], zb)
    for j in range(NSLABCH):
        pltpu.async_copy(zb, acc.at[pl.ds(r0 + j * B, B)], zsem)
    for j in range(NSLABCH):
        pltpu.make_async_copy(zb, acc.at[pl.ds(r0, B)], zsem).wait()
    plsc.subcore_barrier()

    def s_idx(i, b):
        pltpu.async_copy(srcp.at[pl.ds(base_w + i * B, B)], srcv2.at[b],
                         isem.at[b])
        pltpu.async_copy(dstp.at[pl.ds(base_w + i * B, B)], dstv2.at[b],
                         isem.at[b])

    def w_idx(b):
        pltpu.make_async_copy(srcp.at[pl.ds(0, B)], srcv2.at[b],
                              isem.at[b]).wait()
        pltpu.make_async_copy(dstp.at[pl.ds(0, B)], dstv2.at[b],
                              isem.at[b]).wait()

    def s_gather(b, b4):
        pltpu.async_copy(u16.at[srcv2.at[b]], rows2.at[b4], gsem.at[b4])

    def w_gather(b4):
        pltpu.make_async_copy(u16.at[pl.ds(0, B)], rows2.at[b4],
                              gsem.at[b4]).wait()

    def s_scatter(b, b4):
        pltpu.async_copy(rows2.at[b4], acc.at[dstv2.at[b]], ssem.at[b4],
                         add=True)

    def w_scatter(b4):
        pltpu.make_async_copy(u16.at[pl.ds(0, B)], rows2.at[b4],
                              ssem.at[b4]).wait()

    for i in range(3):
        s_idx(i, i)
    for i in range(8):
        b4 = i % 4
        if i >= 4:
            w_scatter(b4)
        w_idx(i)
        s_gather(i, b4)
        if i >= 1:
            w_gather((i - 1) % 4)
            s_scatter(i - 1, (i - 1) % 4)
        s_idx(i + 3, (i + 3) % 8)

    def outer(o, carry):
        for b in range(8):
            i = o * 8 + b
            b4 = b % 4
            w_scatter(b4)
            w_idx(b)
            s_gather(b, b4)
            pb, pb4 = (b - 1) % 8, (b - 1) % 4
            w_gather(pb4)
            s_scatter(pb, pb4)

            @pl.when(i + 3 < NCHUNK)
            def _():
                s_idx(i + 3, (b + 3) % 8)
        return carry

    lax.fori_loop(1, NCHUNK // 8, outer, 0)

    w_gather(3)
    s_scatter(7, 3)
    for b4 in range(4):
        w_scatter(b4)
    plsc.subcore_barrier()

    for j in range(NSLABCH):
        bj = j % 2
        if j >= 2:
            pltpu.make_async_copy(u16.at[pl.ds(0, B)], zb, osem.at[bj]).wait()
        pltpu.sync_copy(acc.at[pl.ds(r0 + j * B, B)], rows2.at[bj])
        pltpu.async_copy(rows2.at[bj], out.at[c, pl.ds(r0 + j * B, B)],
                         osem.at[bj])
    for bj in ((NSLABCH - 2) % 2, (NSLABCH - 1) % 2):
        pltpu.make_async_copy(u16.at[pl.ds(0, B)], zb, osem.at[bj]).wait()


def _tc_a_body(zp, x8b, g2, r8, b1, u2o, dvo):
    Z2 = zp[0] + zp[1]
    m12 = lax.broadcasted_iota(jnp.int32, (1, 16), 1) == 12
    cnt = jnp.sum(jnp.where(m12, Z2, 0.0), axis=1, keepdims=True)
    rcp = 1.0 / jnp.maximum(cnt, 1.0)
    agg = jnp.dot(Z2, g2[...], preferred_element_type=jnp.float32) * rcp
    xr = jnp.dot(x8b[...], r8[...], preferred_element_type=jnp.float32)
    h1 = jnp.maximum(agg + xr + b1[0:1, :], 0.0)
    dinv = lax.rsqrt(cnt + 1.0)
    gidx = lax.broadcasted_iota(jnp.int32, (NB, 1), 0) + pl.program_id(0) * NB
    mask = gidx < N
    u2o[...] = jnp.where(mask, h1 * dinv, 0.0)
    dvo[...] = jnp.broadcast_to(jnp.where(mask, dinv, 1.0), (NB, 16))


def _tc_a(zpair, x8, g2p, root8, b1p):
    return pl.pallas_call(
        _tc_a_body,
        grid=(NGRID,),
        in_specs=[
            pl.BlockSpec((NC, NB, 16), lambda i: (0, i, 0)),
            pl.BlockSpec((NB, 8), lambda i: (i, 0)),
            pl.BlockSpec((16, 16), lambda i: (0, 0)),
            pl.BlockSpec((8, 16), lambda i: (0, 0)),
            pl.BlockSpec((8, 16), lambda i: (0, 0)),
        ],
        out_specs=[
            pl.BlockSpec((NB, 16), lambda i: (i, 0)),
            pl.BlockSpec((NB, 16), lambda i: (i, 0)),
        ],
        out_shape=[
            jax.ShapeDtypeStruct((NP, 16), jnp.float32),
            jax.ShapeDtypeStruct((NP, 16), jnp.float32),
        ],
    )(zpair, x8, g2p, root8, b1p)


def _tc_b_body(sp, ub, dv, w, b, uo, *, relu):
    P = dv[...] * (sp[0] + sp[1] + ub[...])
    h = jnp.dot(P, w[...], preferred_element_type=jnp.float32) + b[0:1, :]
    if relu:
        h = jnp.maximum(h, 0.0)
    gidx = lax.broadcasted_iota(jnp.int32, (NB, 1), 0) + pl.program_id(0) * NB
    uo[...] = jnp.where(gidx < N, h * dv[...], 0.0)


def _tc_b(spair, u, dv16, wp, bp, relu):
    return pl.pallas_call(
        functools.partial(_tc_b_body, relu=relu),
        grid=(NGRID,),
        in_specs=[
            pl.BlockSpec((NC, NB, 16), lambda i: (0, i, 0)),
            pl.BlockSpec((NB, 16), lambda i: (i, 0)),
            pl.BlockSpec((NB, 16), lambda i: (i, 0)),
            pl.BlockSpec((16, 16), lambda i: (0, 0)),
            pl.BlockSpec((8, 16), lambda i: (0, 0)),
        ],
        out_specs=pl.BlockSpec((NB, 16), lambda i: (i, 0)),
        out_shape=jax.ShapeDtypeStruct((NP, 16), jnp.float32),
    )(spair, u, dv16, wp, bp)


def _tc_c_body(sp, ub, dv, w3, b3, bb, wl, bl, outo, acc):
    i = pl.program_id(0)

    @pl.when(i == 0)
    def _init():
        acc[...] = jnp.zeros((G, 32), jnp.float32)

    P = dv[...] * (sp[0] + sp[1] + ub[...])
    h3 = jnp.dot(P, w3[...], preferred_element_type=jnp.float32) + b3[0:1, :]
    oh = (bb[...] == lax.broadcasted_iota(jnp.int32, (1, G), 1)).astype(jnp.float32)
    acc[:, 0:16] += lax.dot_general(
        oh, h3, (((0,), (0,)), ((), ())), preferred_element_type=jnp.float32)
    acc[:, 16:17] += jnp.sum(oh, axis=0)[:, None]

    @pl.when(i == NGRID - 1)
    def _fin():
        gc = jnp.maximum(acc[:, 16:17], 1.0)
        pooled = acc[:, 0:16] / gc
        logits = jnp.dot(pooled, wl[...], preferred_element_type=jnp.float32) + bl[0:1, :]
        colmask = lax.broadcasted_iota(jnp.int32, (1, 8), 1) < C
        lm = jnp.where(colmask, logits, -1e30)
        mx = jnp.max(lm, axis=1, keepdims=True)
        ex = jnp.where(colmask, jnp.exp(lm - mx), 0.0)
        outo[...] = ex / jnp.sum(ex, axis=1, keepdims=True)


def _tc_c(spair, u3, dv16, w3p, b3p, batchcol, wlp, blp):
    return pl.pallas_call(
        _tc_c_body,
        grid=(NGRID,),
        in_specs=[
            pl.BlockSpec((NC, NB, 16), lambda i: (0, i, 0)),
            pl.BlockSpec((NB, 16), lambda i: (i, 0)),
            pl.BlockSpec((NB, 16), lambda i: (i, 0)),
            pl.BlockSpec((16, 16), lambda i: (0, 0)),
            pl.BlockSpec((8, 16), lambda i: (0, 0)),
            pl.BlockSpec((NB, 1), lambda i: (i, 0)),
            pl.BlockSpec((16, 8), lambda i: (0, 0)),
            pl.BlockSpec((8, 8), lambda i: (0, 0)),
        ],
        out_specs=pl.BlockSpec((G, 8), lambda i: (0, 0)),
        out_shape=jax.ShapeDtypeStruct((G, 8), jnp.float32),
        scratch_shapes=[pltpu.VMEM((G, 32), jnp.float32)],
    )(spair, u3, dv16, w3p, b3p, batchcol, wlp, blp)


def kernel(x, edge_index, edge_attr, batch, mu, sigma, g, root,
           b1, W2, b2, W3, b3, Wl, bl):
    f32 = jnp.float32
    src = edge_index[0]
    dst = edge_index[1]

    pad_e = EP - E
    srcp = jnp.concatenate([src, jnp.full((pad_e,), NP - 1, jnp.int32)])
    dstp = jnp.concatenate([dst, jnp.full((pad_e,), NP - 1, jnp.int32)])
    eaT = jnp.concatenate(
        [edge_attr.T, jnp.zeros((D, pad_e), f32)], axis=1).reshape(-1)
    x8 = jnp.zeros((NP, 8), f32).at[:N, :D].set(x)
    zeros16 = jnp.zeros((NP, 16), f32)

    a = -0.5 / (1e-15 + sigma * sigma)
    coefs = jnp.concatenate([
        jnp.repeat(a.reshape(-1), 16),
        jnp.repeat(mu.reshape(-1), 16),
    ]).astype(f32)

    g2p = jnp.zeros((16, 16), f32).at[:K * D, :].set(
        g.reshape(D, K, H).transpose(1, 0, 2).reshape(K * D, H))
    root8 = jnp.zeros((8, 16), f32).at[:D, :].set(root)
    b1p = jnp.broadcast_to(b1, (8, 16)).astype(f32)
    w2p = W2.astype(f32)
    b2p = jnp.broadcast_to(b2, (8, 16)).astype(f32)
    w3p = W3.astype(f32)
    b3p = jnp.broadcast_to(b3, (8, 16)).astype(f32)
    wlp = jnp.zeros((16, 8), f32).at[:, :C].set(Wl)
    blp = jnp.zeros((8, 8), f32).at[:, :C].set(jnp.broadcast_to(bl, (8, C)))
    batchcol = jnp.concatenate(
        [batch, jnp.full((NP - N,), G, jnp.int32)]).reshape(NP, 1)

    zpair = _sc_gmm(eaT, srcp, dstp, x8, coefs, zeros16)
    u2, dv16 = _tc_a(zpair, x8, g2p, root8, b1p)
    s2 = _sc_gcn(u2, srcp, dstp, zeros16)
    u3 = _tc_b(s2, u2, dv16, w2p, b2p, relu=True)
    s3 = _sc_gcn(u3, srcp, dstp, zeros16)
    out8 = _tc_c(s3, u3, dv16, w3p, b3p, batchcol, wlp, blp)
    return out8[:, :C]


# speedup vs baseline: 39.1688x; 1.3593x over previous
import functools

import jax
import jax.numpy as jnp
from jax import lax
from jax.experimental import pallas as pl
from jax.experimental.pallas import tpu as pltpu
from jax.experimental.pallas import tpu_sc as plsc

N = 50000
E = 800000
H = 16
K = 4
D = 3
G = 64
C = 2

NC = 2
NS = 16
NW = NC * NS

NP = 51200
SLAB = NP // NS
EP = 819200
EPW = EP // NW
B = 128
NCHUNK = EPW // B
NSLABCH = SLAB // B

NB = 1024
NGRID = NP // NB

_mesh = plsc.VectorSubcoreMesh(
    core_axis_name="c", subcore_axis_name="s", num_cores=NC, num_subcores=NS
)


def _lane_const(v, dtype=jnp.float32):
    return jnp.full((16,), v, dtype)


@functools.partial(
    pl.kernel,
    out_type=jax.ShapeDtypeStruct((NC, NP, 16), jnp.float32),
    mesh=_mesh,
    compiler_params=pltpu.CompilerParams(needs_layout_passes=False, use_tc_tiling_on_sc=False),
    scratch_types=[
        pltpu.VMEM_SHARED((NP, 16), jnp.float32),
        pltpu.VMEM((8, B), jnp.int32),
        pltpu.VMEM((8, B), jnp.int32),
        pltpu.VMEM((8 * 3 * B,), jnp.float32),
        pltpu.VMEM((2, B, 8), jnp.float32),
        pltpu.VMEM((2, B, 16), jnp.float32),
        pltpu.VMEM((384,), jnp.float32),
        pltpu.SemaphoreType.DMA((8,)),
        pltpu.SemaphoreType.DMA((2,)),
        pltpu.SemaphoreType.DMA((2,)),
        pltpu.SemaphoreType.DMA((2,)),
        pltpu.SemaphoreType.DMA,
    ],
)
def _sc_gmm(eaT, srcp, dstp, x8, coefs, zeros16, out,
            zacc, srcv2, dstv2, eav2, xrows2, zbuf2, cvm,
            isem, gsem, ssem, osem, zsem):
    c = lax.axis_index("c")
    s = lax.axis_index("s")
    w = c * NS + s
    r0 = s * SLAB
    base_w = w * EPW

    pltpu.sync_copy(zeros16.at[pl.ds(0, B)], zbuf2.at[0])
    for j in range(NSLABCH):
        pltpu.async_copy(zbuf2.at[0], zacc.at[pl.ds(r0 + j * B, B)], zsem)
    pltpu.sync_copy(coefs, cvm)
    for j in range(NSLABCH):
        pltpu.make_async_copy(zbuf2.at[0], zacc.at[pl.ds(r0, B)], zsem).wait()
    for z2 in range(2):
        for j in range(B // 16):
            rows = lax.iota(jnp.int32, 16) + (j * 16)
            plsc.store_scatter(
                zbuf2, [_lane_const(z2, jnp.int32), rows,
                        _lane_const(12, jnp.int32)], _lane_const(1.0))
            for col in (13, 14, 15):
                plsc.store_scatter(
                    zbuf2, [_lane_const(z2, jnp.int32), rows,
                            _lane_const(col, jnp.int32)], _lane_const(0.0))
    plsc.subcore_barrier()

    def s_idx(i, b):
        base = base_w + i * B
        pltpu.async_copy(srcp.at[pl.ds(base, B)], srcv2.at[b], isem.at[b])
        pltpu.async_copy(dstp.at[pl.ds(base, B)], dstv2.at[b], isem.at[b])
        for d in range(3):
            pltpu.async_copy(eaT.at[pl.ds(d * EP + base, B)],
                             eav2.at[pl.ds((b * 3 + d) * B, B)], isem.at[b])

    def w_idx(b):
        pltpu.make_async_copy(srcp.at[pl.ds(0, B)], srcv2.at[b],
                              isem.at[b]).wait()
        pltpu.make_async_copy(dstp.at[pl.ds(0, B)], dstv2.at[b],
                              isem.at[b]).wait()
        for d in range(3):
            pltpu.make_async_copy(eaT.at[pl.ds(0, B)],
                                  eav2.at[pl.ds((b * 3 + d) * B, B)],
                                  isem.at[b]).wait()

    def s_gather(i, b, g2):
        pltpu.async_copy(x8.at[srcv2.at[b]], xrows2.at[g2], gsem.at[g2])

    def w_gather(g2):
        pltpu.make_async_copy(x8.at[pl.ds(0, B)], xrows2.at[g2],
                              gsem.at[g2]).wait()

    def s_scatter(b, z2):
        pltpu.async_copy(zbuf2.at[z2], zacc.at[dstv2.at[b]], ssem.at[z2],
                         add=True)

    def w_scatter(z2):
        pltpu.make_async_copy(zeros16.at[pl.ds(0, B)], zbuf2.at[z2],
                              ssem.at[z2]).wait()

    av = [cvm[pl.ds(t * 16, 16)] for t in range(12)]
    mv = [cvm[pl.ds(192 + t * 16, 16)] for t in range(12)]

    def compute(b, g2, z2):
        for j in range(B // 16):
            rows = lax.iota(jnp.int32, 16) + (j * 16)
            eas = [eav2[pl.ds((b * 3 + d) * B + j * 16, 16)] for d in range(3)]
            xc = [plsc.load_gather(
                      xrows2, [_lane_const(g2, jnp.int32), rows,
                               _lane_const(d, jnp.int32)])
                  for d in range(3)]
            zsel = _lane_const(z2, jnp.int32)
            for k in range(K):
                t = None
                for d in range(3):
                    df = eas[d] - mv[k * 3 + d]
                    term = av[k * 3 + d] * df * df
                    t = term if t is None else t + term
                gk = jnp.exp(t)
                for d in range(3):
                    plsc.store_scatter(
                        zbuf2, [zsel, rows, _lane_const(k * 3 + d, jnp.int32)],
                        gk * xc[d])

    for i in range(4):
        s_idx(i, i)
    w_idx(0)
    s_gather(0, 0, 0)

    def outer(o, carry):
        for b in range(8):
            i = o * 8 + b

            @pl.when(i + 1 < NCHUNK)
            def _():
                w_idx((b + 1) % 8)
                s_gather(i + 1, (b + 1) % 8, (b + 1) % 2)

            @pl.when(i >= 2)
            def _():
                w_scatter(b % 2)

            w_gather(b % 2)
            compute(b, b % 2, b % 2)
            s_scatter(b, b % 2)

            @pl.when(i + 4 < NCHUNK)
            def _():
                s_idx(i + 4, (b + 4) % 8)
        return carry

    lax.fori_loop(0, NCHUNK // 8, outer, 0)

    for z2 in range(2):
        w_scatter(z2)
    plsc.subcore_barrier()

    for j in range(NSLABCH):
        bj = j % 2
        if j >= 2:
            pltpu.make_async_copy(zeros16.at[pl.ds(0, B)], zbuf2.at[bj],
                                  osem.at[bj]).wait()
        pltpu.sync_copy(zacc.at[pl.ds(r0 + j * B, B)], zbuf2.at[bj])
        pltpu.async_copy(zbuf2.at[bj], out.at[c, pl.ds(r0 + j * B, B)],
                         osem.at[bj])
    for bj in ((NSLABCH - 2) % 2, (NSLABCH - 1) % 2):
        pltpu.make_async_copy(zeros16.at[pl.ds(0, B)], zbuf2.at[bj],
                              osem.at[bj]).wait()


@functools.partial(
    pl.kernel,
    out_type=jax.ShapeDtypeStruct((NC, NP, 16), jnp.float32),
    mesh=_mesh,
    compiler_params=pltpu.CompilerParams(needs_layout_passes=False, use_tc_tiling_on_sc=False),
    scratch_types=[
        pltpu.VMEM_SHARED((NP, 16), jnp.float32),
        pltpu.VMEM((8, B), jnp.int32),
        pltpu.VMEM((8, B), jnp.int32),
        pltpu.VMEM((4, B, 16), jnp.float32),
        pltpu.VMEM((B, 16), jnp.float32),
        pltpu.SemaphoreType.DMA((8,)),
        pltpu.SemaphoreType.DMA((4,)),
        pltpu.SemaphoreType.DMA((4,)),
        pltpu.SemaphoreType.DMA((2,)),
        pltpu.SemaphoreType.DMA,
    ],
)
def _sc_gcn(u16, srcp, dstp, zeros16, out,
            acc, srcv2, dstv2, rows2, zb, isem, gsem, ssem, osem, zsem):
    c = lax.axis_index("c")
    s = lax.axis_index("s")
    w = c * NS + s
    r0 = s * SLAB
    base_w = w * EPW

    pltpu.sync_copy(zeros16.at[pl.ds(0, B)], zb)
    for j in range(NSLABCH):
        pltpu.async_copy(zb, acc.at[pl.ds(r0 + j * B, B)], zsem)
    for j in range(NSLABCH):
        pltpu.make_async_copy(zb, acc.at[pl.ds(r0, B)], zsem).wait()
    plsc.subcore_barrier()

    def s_idx(i, b):
        pltpu.async_copy(srcp.at[pl.ds(base_w + i * B, B)], srcv2.at[b],
                         isem.at[b])
        pltpu.async_copy(dstp.at[pl.ds(base_w + i * B, B)], dstv2.at[b],
                         isem.at[b])

    def w_idx(b):
        pltpu.make_async_copy(srcp.at[pl.ds(0, B)], srcv2.at[b],
                              isem.at[b]).wait()
        pltpu.make_async_copy(dstp.at[pl.ds(0, B)], dstv2.at[b],
                              isem.at[b]).wait()

    def s_gather(b, b4):
        pltpu.async_copy(u16.at[srcv2.at[b]], rows2.at[b4], gsem.at[b4])

    def w_gather(b4):
        pltpu.make_async_copy(u16.at[pl.ds(0, B)], rows2.at[b4],
                              gsem.at[b4]).wait()

    def s_scatter(b, b4):
        pltpu.async_copy(rows2.at[b4], acc.at[dstv2.at[b]], ssem.at[b4],
                         add=True)

    def w_scatter(b4):
        pltpu.make_async_copy(u16.at[pl.ds(0, B)], rows2.at[b4],
                              ssem.at[b4]).wait()

    for i in range(3):
        s_idx(i, i)
    for i in range(8):
        b4 = i % 4
        if i >= 4:
            w_scatter(b4)
        w_idx(i)
        s_gather(i, b4)
        if i >= 1:
            w_gather((i - 1) % 4)
            s_scatter(i - 1, (i - 1) % 4)
        s_idx(i + 3, (i + 3) % 8)

    def outer(o, carry):
        for b in range(8):
            i = o * 8 + b
            b4 = b % 4
            w_scatter(b4)
            w_idx(b)
            s_gather(b, b4)
            pb, pb4 = (b - 1) % 8, (b - 1) % 4
            w_gather(pb4)
            s_scatter(pb, pb4)

            @pl.when(i + 3 < NCHUNK)
            def _():
                s_idx(i + 3, (b + 3) % 8)
        return carry

    lax.fori_loop(1, NCHUNK // 8, outer, 0)

    w_gather(3)
    s_scatter(7, 3)
    for b4 in range(4):
        w_scatter(b4)
    plsc.subcore_barrier()

    for j in range(NSLABCH):
        bj = j % 2
        if j >= 2:
            pltpu.make_async_copy(u16.at[pl.ds(0, B)], zb, osem.at[bj]).wait()
        pltpu.sync_copy(acc.at[pl.ds(r0 + j * B, B)], rows2.at[bj])
        pltpu.async_copy(rows2.at[bj], out.at[c, pl.ds(r0 + j * B, B)],
                         osem.at[bj])
    for bj in ((NSLABCH - 2) % 2, (NSLABCH - 1) % 2):
        pltpu.make_async_copy(u16.at[pl.ds(0, B)], zb, osem.at[bj]).wait()


NR = NP // 8
NBP = 640
NGRIDP = NR // NBP


def _node_mask(i):
    rowi = lax.broadcasted_iota(jnp.int32, (NBP, 128), 0)
    coli = lax.broadcasted_iota(jnp.int32, (NBP, 128), 1)
    node = (i * NBP + rowi) * 8 + (coli >> 4)
    return node < N


def _tc_a_body(zp, xp, s12, g2, r8, b1, u2o, dvo):
    Z2 = zp[0] + zp[1]
    cntb = jnp.dot(Z2, s12[...], preferred_element_type=jnp.float32)
    rcp = 1.0 / jnp.maximum(cntb, 1.0)
    agg = jnp.dot(Z2, g2[...], preferred_element_type=jnp.float32) * rcp
    xr = jnp.dot(xp[...], r8[...], preferred_element_type=jnp.float32)
    h1 = jnp.maximum(agg + xr + b1[0:1, :], 0.0)
    dinv = lax.rsqrt(cntb + 1.0)
    mask = _node_mask(pl.program_id(0))
    u2o[...] = jnp.where(mask, h1 * dinv, 0.0)
    dvo[...] = jnp.where(mask, dinv, 1.0)


def _tc_a(zpair, xpk, s12, g2bd, rootbd, b1t):
    return pl.pallas_call(
        _tc_a_body,
        grid=(NGRIDP,),
        in_specs=[
            pl.BlockSpec((NC, NBP, 128), lambda i: (0, i, 0)),
            pl.BlockSpec((NBP, 64), lambda i: (i, 0)),
            pl.BlockSpec((128, 128), lambda i: (0, 0)),
            pl.BlockSpec((128, 128), lambda i: (0, 0)),
            pl.BlockSpec((64, 128), lambda i: (0, 0)),
            pl.BlockSpec((8, 128), lambda i: (0, 0)),
        ],
        out_specs=[
            pl.BlockSpec((NBP, 128), lambda i: (i, 0)),
            pl.BlockSpec((NBP, 128), lambda i: (i, 0)),
        ],
        out_shape=[
            jax.ShapeDtypeStruct((NR, 128), jnp.float32),
            jax.ShapeDtypeStruct((NR, 128), jnp.float32),
        ],
    )(zpair, xpk, s12, g2bd, rootbd, b1t)


def _tc_b_body(sp, ub, dv, w, b, uo, *, relu):
    P = dv[...] * (sp[0] + sp[1] + ub[...])
    h = jnp.dot(P, w[...], preferred_element_type=jnp.float32) + b[0:1, :]
    if relu:
        h = jnp.maximum(h, 0.0)
    uo[...] = jnp.where(_node_mask(pl.program_id(0)), h * dv[...], 0.0)


def _tc_b(spair, u, dvp, wbd, bt, relu):
    return pl.pallas_call(
        functools.partial(_tc_b_body, relu=relu),
        grid=(NGRIDP,),
        in_specs=[
            pl.BlockSpec((NC, NBP, 128), lambda i: (0, i, 0)),
            pl.BlockSpec((NBP, 128), lambda i: (i, 0)),
            pl.BlockSpec((NBP, 128), lambda i: (i, 0)),
            pl.BlockSpec((128, 128), lambda i: (0, 0)),
            pl.BlockSpec((8, 128), lambda i: (0, 0)),
        ],
        out_specs=pl.BlockSpec((NBP, 128), lambda i: (i, 0)),
        out_shape=jax.ShapeDtypeStruct((NR, 128), jnp.float32),
    )(spair, u, dvp, wbd, bt)


def _tc_c_body(sp, ub, dv, w3, b3, bpk, r8sel, wl, bl, outo, p1, cnts):
    i = pl.program_id(0)

    @pl.when(i == 0)
    def _init():
        p1[...] = jnp.zeros((512, 128), jnp.float32)
        cnts[...] = jnp.zeros((512, 8), jnp.float32)

    P = dv[...] * (sp[0] + sp[1] + ub[...])
    h3 = jnp.dot(P, w3[...], preferred_element_type=jnp.float32) + b3[0:1, :]
    bfl = bpk[...].astype(jnp.float32)
    b512 = jnp.dot(bfl, r8sel[...], preferred_element_type=jnp.float32)
    colg = (lax.broadcasted_iota(jnp.int32, (NBP, 512), 1) %
            jnp.int32(G)).astype(jnp.float32)
    ohp = (b512 == colg).astype(jnp.float32)
    p1[...] += lax.dot_general(
        ohp, h3, (((0,), (0,)), ((), ())), preferred_element_type=jnp.float32)
    cnts[...] += lax.dot_general(
        ohp, jnp.ones((NBP, 8), jnp.float32), (((0,), (0,)), ((), ())),
        preferred_element_type=jnp.float32)

    @pl.when(i == NGRIDP - 1)
    def _fin():
        pooled = jnp.zeros((G, 16), jnp.float32)
        gc = jnp.zeros((G, 1), jnp.float32)
        for q in range(8):
            pooled = pooled + p1[q * G:(q + 1) * G, q * 16:(q + 1) * 16]
            gc = gc + cnts[q * G:(q + 1) * G, 0:1]
        pooled = pooled / jnp.maximum(gc, 1.0)
        logits = jnp.dot(pooled, wl[...],
                         preferred_element_type=jnp.float32) + bl[0:1, :]
        colmask = lax.broadcasted_iota(jnp.int32, (1, 8), 1) < C
        lm = jnp.where(colmask, logits, -1e30)
        mx = jnp.max(lm, axis=1, keepdims=True)
        ex = jnp.where(colmask, jnp.exp(lm - mx), 0.0)
        outo[...] = ex / jnp.sum(ex, axis=1, keepdims=True)


def _tc_c(spair, u3, dvp, w3bd, b3t, batchpk, r8sel, wlp, blp):
    return pl.pallas_call(
        _tc_c_body,
        grid=(NGRIDP,),
        in_specs=[
            pl.BlockSpec((NC, NBP, 128), lambda i: (0, i, 0)),
            pl.BlockSpec((NBP, 128), lambda i: (i, 0)),
            pl.BlockSpec((NBP, 128), lambda i: (i, 0)),
            pl.BlockSpec((128, 128), lambda i: (0, 0)),
            pl.BlockSpec((8, 128), lambda i: (0, 0)),
            pl.BlockSpec((NBP, 8), lambda i: (i, 0)),
            pl.BlockSpec((8, 512), lambda i: (0, 0)),
            pl.BlockSpec((16, 8), lambda i: (0, 0)),
            pl.BlockSpec((8, 8), lambda i: (0, 0)),
        ],
        out_specs=pl.BlockSpec((G, 8), lambda i: (0, 0)),
        out_shape=jax.ShapeDtypeStruct((G, 8), jnp.float32),
        scratch_shapes=[pltpu.VMEM((512, 128), jnp.float32),
                        pltpu.VMEM((512, 8), jnp.float32)],
    )(spair, u3, dvp, w3bd, b3t, batchpk, r8sel, wlp, blp)


def kernel(x, edge_index, edge_attr, batch, mu, sigma, g, root,
           b1, W2, b2, W3, b3, Wl, bl):
    f32 = jnp.float32
    src = edge_index[0]
    dst = edge_index[1]

    pad_e = EP - E
    srcp = jnp.concatenate([src, jnp.full((pad_e,), NP - 1, jnp.int32)])
    dstp = jnp.concatenate([dst, jnp.full((pad_e,), NP - 1, jnp.int32)])
    eaT = jnp.concatenate(
        [edge_attr.T, jnp.zeros((D, pad_e), f32)], axis=1).reshape(-1)
    x8 = jnp.zeros((NP, 8), f32).at[:N, :D].set(x)
    zeros16 = jnp.zeros((NP, 16), f32)

    a = -0.5 / (1e-15 + sigma * sigma)
    coefs = jnp.concatenate([
        jnp.repeat(a.reshape(-1), 16),
        jnp.repeat(mu.reshape(-1), 16),
    ]).astype(f32)

    eye8 = jnp.eye(8, dtype=f32)
    g2p = jnp.zeros((16, 16), f32).at[:K * D, :].set(
        g.reshape(D, K, H).transpose(1, 0, 2).reshape(K * D, H))
    root8 = jnp.zeros((8, 16), f32).at[:D, :].set(root)
    g2bd = jnp.kron(eye8, g2p)
    rootbd = jnp.kron(eye8, root8)
    w2bd = jnp.kron(eye8, W2.astype(f32))
    w3bd = jnp.kron(eye8, W3.astype(f32))
    b1t = jnp.broadcast_to(jnp.tile(b1, 8), (8, 128)).astype(f32)
    b2t = jnp.broadcast_to(jnp.tile(b2, 8), (8, 128)).astype(f32)
    b3t = jnp.broadcast_to(jnp.tile(b3, 8), (8, 128)).astype(f32)
    ci = jnp.arange(128)
    s12 = (ci[:, None] == (ci[None, :] // 16) * 16 + 12).astype(f32)
    r8sel = (jnp.arange(512)[None, :] // G ==
             jnp.arange(8)[:, None]).astype(f32)
    wlp = jnp.zeros((16, 8), f32).at[:, :C].set(Wl)
    blp = jnp.zeros((8, 8), f32).at[:, :C].set(jnp.broadcast_to(bl, (8, C)))
    batchpk = jnp.concatenate(
        [batch, jnp.full((NP - N,), G, jnp.int32)]).reshape(NR, 8)
    xpk = x8.reshape(NR, 64)

    zpair = _sc_gmm(eaT, srcp, dstp, x8, coefs, zeros16)
    u2, dvp = _tc_a(zpair.reshape(NC, NR, 128), xpk, s12, g2bd, rootbd, b1t)
    s2 = _sc_gcn(u2.reshape(NP, 16), srcp, dstp, zeros16)
    u3 = _tc_b(s2.reshape(NC, NR, 128), u2, dvp, w2bd, b2t, relu=True)
    s3 = _sc_gcn(u3.reshape(NP, 16), srcp, dstp, zeros16)
    out8 = _tc_c(s3.reshape(NC, NR, 128), u3, dvp, w3bd, b3t, batchpk,
                 r8sel, wlp, blp)
    return out8[:, :C]


# speedup vs baseline: 40.3274x; 1.0296x over previous
import functools

import jax
import jax.numpy as jnp
from jax import lax
from jax.experimental import pallas as pl
from jax.experimental.pallas import tpu as pltpu
from jax.experimental.pallas import tpu_sc as plsc

N = 50000
E = 800000
H = 16
K = 4
D = 3
G = 64
C = 2

NC = 2
NS = 16
NW = NC * NS

NP = 51200
SLAB = NP // NS
EP = 819200
EPW = EP // NW
B = 128
NCHUNK = EPW // B
NSLABCH = SLAB // B

NB = 1024
NGRID = NP // NB

_mesh = plsc.VectorSubcoreMesh(
    core_axis_name="c", subcore_axis_name="s", num_cores=NC, num_subcores=NS
)


def _lane_const(v, dtype=jnp.float32):
    return jnp.full((16,), v, dtype)


@functools.partial(
    pl.kernel,
    out_type=jax.ShapeDtypeStruct((NC, NP, 16), jnp.float32),
    mesh=_mesh,
    compiler_params=pltpu.CompilerParams(needs_layout_passes=False, use_tc_tiling_on_sc=False),
    scratch_types=[
        pltpu.VMEM_SHARED((NP, 16), jnp.float32),
        pltpu.VMEM((8, 2, B), jnp.int32),
        pltpu.VMEM((8 * 3 * B,), jnp.float32),
        pltpu.VMEM((2, B, 8), jnp.float32),
        pltpu.VMEM((2, B, 16), jnp.float32),
        pltpu.VMEM((384,), jnp.float32),
        pltpu.SemaphoreType.DMA((8,)),
        pltpu.SemaphoreType.DMA((2,)),
        pltpu.SemaphoreType.DMA((2,)),
        pltpu.SemaphoreType.DMA((2,)),
        pltpu.SemaphoreType.DMA,
    ],
)
def _sc_gmm(eaC, sdp, x8, coefs, zeros16, out,
            zacc, sdv2, eav2, xrows2, zbuf2, cvm,
            isem, gsem, ssem, osem, zsem):
    c = lax.axis_index("c")
    s = lax.axis_index("s")
    w = c * NS + s
    r0 = s * SLAB

    pltpu.sync_copy(zeros16.at[pl.ds(0, B)], zbuf2.at[0])
    for j in range(NSLABCH):
        pltpu.async_copy(zbuf2.at[0], zacc.at[pl.ds(r0 + j * B, B)], zsem)
    pltpu.sync_copy(coefs, cvm)
    for j in range(NSLABCH):
        pltpu.make_async_copy(zbuf2.at[0], zacc.at[pl.ds(r0, B)], zsem).wait()
    for z2 in range(2):
        for j in range(B // 16):
            rows = lax.iota(jnp.int32, 16) + (j * 16)
            plsc.store_scatter(
                zbuf2, [_lane_const(z2, jnp.int32), rows,
                        _lane_const(12, jnp.int32)], _lane_const(1.0))
            for col in (13, 14, 15):
                plsc.store_scatter(
                    zbuf2, [_lane_const(z2, jnp.int32), rows,
                            _lane_const(col, jnp.int32)], _lane_const(0.0))
    plsc.subcore_barrier()

    def s_idx(i, b):
        ch = w * NCHUNK + i
        pltpu.async_copy(sdp.at[ch], sdv2.at[b], isem.at[b])
        pltpu.async_copy(eaC.at[pl.ds(ch * 3 * B, 3 * B)],
                         eav2.at[pl.ds(b * 3 * B, 3 * B)], isem.at[b])

    def w_idx(b):
        pltpu.make_async_copy(sdp.at[0], sdv2.at[b], isem.at[b]).wait()
        pltpu.make_async_copy(eaC.at[pl.ds(0, 3 * B)],
                              eav2.at[pl.ds(b * 3 * B, 3 * B)],
                              isem.at[b]).wait()

    def s_gather(i, b, g2):
        pltpu.async_copy(x8.at[sdv2.at[b, 0]], xrows2.at[g2], gsem.at[g2])

    def w_gather(g2):
        pltpu.make_async_copy(x8.at[pl.ds(0, B)], xrows2.at[g2],
                              gsem.at[g2]).wait()

    def s_scatter(b, z2):
        pltpu.async_copy(zbuf2.at[z2], zacc.at[sdv2.at[b, 1]], ssem.at[z2],
                         add=True)

    def w_scatter(z2):
        pltpu.make_async_copy(zeros16.at[pl.ds(0, B)], zbuf2.at[z2],
                              ssem.at[z2]).wait()

    av = [cvm[pl.ds(t * 16, 16)] for t in range(12)]
    mv = [cvm[pl.ds(192 + t * 16, 16)] for t in range(12)]

    def compute(b, g2, z2):
        for j in range(B // 16):
            rows = lax.iota(jnp.int32, 16) + (j * 16)
            eas = [eav2[pl.ds((b * 3 + d) * B + j * 16, 16)] for d in range(3)]
            xc = [plsc.load_gather(
                      xrows2, [_lane_const(g2, jnp.int32), rows,
                               _lane_const(d, jnp.int32)])
                  for d in range(3)]
            zsel = _lane_const(z2, jnp.int32)
            for k in range(K):
                t = None
                for d in range(3):
                    df = eas[d] - mv[k * 3 + d]
                    term = av[k * 3 + d] * df * df
                    t = term if t is None else t + term
                gk = jnp.exp(t)
                for d in range(3):
                    plsc.store_scatter(
                        zbuf2, [zsel, rows, _lane_const(k * 3 + d, jnp.int32)],
                        gk * xc[d])

    for i in range(4):
        s_idx(i, i)
    w_idx(0)
    s_gather(0, 0, 0)

    def outer(o, carry):
        for b in range(8):
            i = o * 8 + b

            @pl.when(i + 1 < NCHUNK)
            def _():
                w_idx((b + 1) % 8)
                s_gather(i + 1, (b + 1) % 8, (b + 1) % 2)

            @pl.when(i >= 2)
            def _():
                w_scatter(b % 2)

            w_gather(b % 2)
            compute(b, b % 2, b % 2)
            s_scatter(b, b % 2)

            @pl.when(i + 4 < NCHUNK)
            def _():
                s_idx(i + 4, (b + 4) % 8)
        return carry

    lax.fori_loop(0, NCHUNK // 8, outer, 0)

    for z2 in range(2):
        w_scatter(z2)
    plsc.subcore_barrier()

    for j in range(NSLABCH):
        bj = j % 2
        if j >= 2:
            pltpu.make_async_copy(zeros16.at[pl.ds(0, B)], zbuf2.at[bj],
                                  osem.at[bj]).wait()
        pltpu.sync_copy(zacc.at[pl.ds(r0 + j * B, B)], zbuf2.at[bj])
        pltpu.async_copy(zbuf2.at[bj], out.at[c, pl.ds(r0 + j * B, B)],
                         osem.at[bj])
    for bj in ((NSLABCH - 2) % 2, (NSLABCH - 1) % 2):
        pltpu.make_async_copy(zeros16.at[pl.ds(0, B)], zbuf2.at[bj],
                              osem.at[bj]).wait()


@functools.partial(
    pl.kernel,
    out_type=jax.ShapeDtypeStruct((NC, NP, 16), jnp.float32),
    mesh=_mesh,
    compiler_params=pltpu.CompilerParams(needs_layout_passes=False, use_tc_tiling_on_sc=False),
    scratch_types=[
        pltpu.VMEM_SHARED((NP, 16), jnp.float32),
        pltpu.VMEM((8, 2, B), jnp.int32),
        pltpu.VMEM((4, B, 16), jnp.float32),
        pltpu.VMEM((B, 16), jnp.float32),
        pltpu.SemaphoreType.DMA((8,)),
        pltpu.SemaphoreType.DMA((4,)),
        pltpu.SemaphoreType.DMA((4,)),
        pltpu.SemaphoreType.DMA((2,)),
        pltpu.SemaphoreType.DMA,
    ],
)
def _sc_gcn(u16, sdp, zeros16, out,
            acc, sdv2, rows2, zb, isem, gsem, ssem, osem, zsem):
    c = lax.axis_index("c")
    s = lax.axis_index("s")
    w = c * NS + s
    r0 = s * SLAB

    pltpu.sync_copy(zeros16.at[pl.ds(0, B)], zb)
    for j in range(NSLABCH):
        pltpu.async_copy(zb, acc.at[pl.ds(r0 + j * B, B)], zsem)
    for j in range(NSLABCH):
        pltpu.make_async_copy(zb, acc.at[pl.ds(r0, B)], zsem).wait()
    plsc.subcore_barrier()

    def s_idx(i, b):
        pltpu.async_copy(sdp.at[w * NCHUNK + i], sdv2.at[b], isem.at[b])

    def w_idx(b):
        pltpu.make_async_copy(sdp.at[0], sdv2.at[b], isem.at[b]).wait()

    def s_gather(b, b4):
        pltpu.async_copy(u16.at[sdv2.at[b, 0]], rows2.at[b4], gsem.at[b4])

    def w_gather(b4):
        pltpu.make_async_copy(u16.at[pl.ds(0, B)], rows2.at[b4],
                              gsem.at[b4]).wait()

    def s_scatter(b, b4):
        pltpu.async_copy(rows2.at[b4], acc.at[sdv2.at[b, 1]], ssem.at[b4],
                         add=True)

    def w_scatter(b4):
        pltpu.make_async_copy(u16.at[pl.ds(0, B)], rows2.at[b4],
                              ssem.at[b4]).wait()

    for i in range(3):
        s_idx(i, i)
    for i in range(8):
        b4 = i % 4
        if i >= 4:
            w_scatter(b4)
        w_idx(i)
        s_gather(i, b4)
        if i >= 1:
            w_gather((i - 1) % 4)
            s_scatter(i - 1, (i - 1) % 4)
        s_idx(i + 3, (i + 3) % 8)

    def outer(o, carry):
        for b in range(8):
            i = o * 8 + b
            b4 = b % 4
            w_scatter(b4)
            w_idx(b)
            s_gather(b, b4)
            pb, pb4 = (b - 1) % 8, (b - 1) % 4
            w_gather(pb4)
            s_scatter(pb, pb4)

            @pl.when(i + 3 < NCHUNK)
            def _():
                s_idx(i + 3, (b + 3) % 8)
        return carry

    lax.fori_loop(1, NCHUNK // 8, outer, 0)

    w_gather(3)
    s_scatter(7, 3)
    for b4 in range(4):
        w_scatter(b4)
    plsc.subcore_barrier()

    for j in range(NSLABCH):
        bj = j % 2
        if j >= 2:
            pltpu.make_async_copy(u16.at[pl.ds(0, B)], zb, osem.at[bj]).wait()
        pltpu.sync_copy(acc.at[pl.ds(r0 + j * B, B)], rows2.at[bj])
        pltpu.async_copy(rows2.at[bj], out.at[c, pl.ds(r0 + j * B, B)],
                         osem.at[bj])
    for bj in ((NSLABCH - 2) % 2, (NSLABCH - 1) % 2):
        pltpu.make_async_copy(u16.at[pl.ds(0, B)], zb, osem.at[bj]).wait()


NR = NP // 8
NBP = 640
NGRIDP = NR // NBP


def _node_mask(i):
    rowi = lax.broadcasted_iota(jnp.int32, (NBP, 128), 0)
    coli = lax.broadcasted_iota(jnp.int32, (NBP, 128), 1)
    node = (i * NBP + rowi) * 8 + (coli >> 4)
    return node < N


def _tc_a_body(zp, xp, s12, g2, r8, b1, u2o, dvo):
    Z2 = zp[0] + zp[1]
    cntb = jnp.dot(Z2, s12[...], preferred_element_type=jnp.float32)
    rcp = 1.0 / jnp.maximum(cntb, 1.0)
    agg = jnp.dot(Z2, g2[...], preferred_element_type=jnp.float32) * rcp
    xr = jnp.dot(xp[...], r8[...], preferred_element_type=jnp.float32)
    h1 = jnp.maximum(agg + xr + b1[0:1, :], 0.0)
    dinv = lax.rsqrt(cntb + 1.0)
    mask = _node_mask(pl.program_id(0))
    u2o[...] = jnp.where(mask, h1 * dinv, 0.0)
    dvo[...] = jnp.where(mask, dinv, 1.0)


def _tc_a(zpair, xpk, s12, g2bd, rootbd, b1t):
    return pl.pallas_call(
        _tc_a_body,
        grid=(NGRIDP,),
        in_specs=[
            pl.BlockSpec((NC, NBP, 128), lambda i: (0, i, 0)),
            pl.BlockSpec((NBP, 64), lambda i: (i, 0)),
            pl.BlockSpec((128, 128), lambda i: (0, 0)),
            pl.BlockSpec((128, 128), lambda i: (0, 0)),
            pl.BlockSpec((64, 128), lambda i: (0, 0)),
            pl.BlockSpec((8, 128), lambda i: (0, 0)),
        ],
        out_specs=[
            pl.BlockSpec((NBP, 128), lambda i: (i, 0)),
            pl.BlockSpec((NBP, 128), lambda i: (i, 0)),
        ],
        out_shape=[
            jax.ShapeDtypeStruct((NR, 128), jnp.float32),
            jax.ShapeDtypeStruct((NR, 128), jnp.float32),
        ],
    )(zpair, xpk, s12, g2bd, rootbd, b1t)


def _tc_b_body(sp, ub, dv, w, b, uo, *, relu):
    P = dv[...] * (sp[0] + sp[1] + ub[...])
    h = jnp.dot(P, w[...], preferred_element_type=jnp.float32) + b[0:1, :]
    if relu:
        h = jnp.maximum(h, 0.0)
    uo[...] = jnp.where(_node_mask(pl.program_id(0)), h * dv[...], 0.0)


def _tc_b(spair, u, dvp, wbd, bt, relu):
    return pl.pallas_call(
        functools.partial(_tc_b_body, relu=relu),
        grid=(NGRIDP,),
        in_specs=[
            pl.BlockSpec((NC, NBP, 128), lambda i: (0, i, 0)),
            pl.BlockSpec((NBP, 128), lambda i: (i, 0)),
            pl.BlockSpec((NBP, 128), lambda i: (i, 0)),
            pl.BlockSpec((128, 128), lambda i: (0, 0)),
            pl.BlockSpec((8, 128), lambda i: (0, 0)),
        ],
        out_specs=pl.BlockSpec((NBP, 128), lambda i: (i, 0)),
        out_shape=jax.ShapeDtypeStruct((NR, 128), jnp.float32),
    )(spair, u, dvp, wbd, bt)


def _tc_c_body(sp, ub, dv, w3, b3, bpk, r8sel, wl, bl, outo, p1, cnts):
    i = pl.program_id(0)

    @pl.when(i == 0)
    def _init():
        p1[...] = jnp.zeros((512, 128), jnp.float32)
        cnts[...] = jnp.zeros((512, 8), jnp.float32)

    P = dv[...] * (sp[0] + sp[1] + ub[...])
    h3 = jnp.dot(P, w3[...], preferred_element_type=jnp.float32) + b3[0:1, :]
    bfl = bpk[...].astype(jnp.float32)
    b512 = jnp.dot(bfl, r8sel[...], preferred_element_type=jnp.float32)
    colg = (lax.broadcasted_iota(jnp.int32, (NBP, 512), 1) %
            jnp.int32(G)).astype(jnp.float32)
    ohp = (b512 == colg).astype(jnp.float32)
    p1[...] += lax.dot_general(
        ohp, h3, (((0,), (0,)), ((), ())), preferred_element_type=jnp.float32)
    cnts[...] += lax.dot_general(
        ohp, jnp.ones((NBP, 8), jnp.float32), (((0,), (0,)), ((), ())),
        preferred_element_type=jnp.float32)

    @pl.when(i == NGRIDP - 1)
    def _fin():
        pooled = jnp.zeros((G, 16), jnp.float32)
        gc = jnp.zeros((G, 1), jnp.float32)
        for q in range(8):
            pooled = pooled + p1[q * G:(q + 1) * G, q * 16:(q + 1) * 16]
            gc = gc + cnts[q * G:(q + 1) * G, 0:1]
        pooled = pooled / jnp.maximum(gc, 1.0)
        logits = jnp.dot(pooled, wl[...],
                         preferred_element_type=jnp.float32) + bl[0:1, :]
        colmask = lax.broadcasted_iota(jnp.int32, (1, 8), 1) < C
        lm = jnp.where(colmask, logits, -1e30)
        mx = jnp.max(lm, axis=1, keepdims=True)
        ex = jnp.where(colmask, jnp.exp(lm - mx), 0.0)
        outo[...] = ex / jnp.sum(ex, axis=1, keepdims=True)


def _tc_c(spair, u3, dvp, w3bd, b3t, batchpk, r8sel, wlp, blp):
    return pl.pallas_call(
        _tc_c_body,
        grid=(NGRIDP,),
        in_specs=[
            pl.BlockSpec((NC, NBP, 128), lambda i: (0, i, 0)),
            pl.BlockSpec((NBP, 128), lambda i: (i, 0)),
            pl.BlockSpec((NBP, 128), lambda i: (i, 0)),
            pl.BlockSpec((128, 128), lambda i: (0, 0)),
            pl.BlockSpec((8, 128), lambda i: (0, 0)),
            pl.BlockSpec((NBP, 8), lambda i: (i, 0)),
            pl.BlockSpec((8, 512), lambda i: (0, 0)),
            pl.BlockSpec((16, 8), lambda i: (0, 0)),
            pl.BlockSpec((8, 8), lambda i: (0, 0)),
        ],
        out_specs=pl.BlockSpec((G, 8), lambda i: (0, 0)),
        out_shape=jax.ShapeDtypeStruct((G, 8), jnp.float32),
        scratch_shapes=[pltpu.VMEM((512, 128), jnp.float32),
                        pltpu.VMEM((512, 8), jnp.float32)],
    )(spair, u3, dvp, w3bd, b3t, batchpk, r8sel, wlp, blp)


def kernel(x, edge_index, edge_attr, batch, mu, sigma, g, root,
           b1, W2, b2, W3, b3, Wl, bl):
    f32 = jnp.float32
    src = edge_index[0]
    dst = edge_index[1]

    pad_e = EP - E
    srcp = jnp.concatenate([src, jnp.full((pad_e,), NP - 1, jnp.int32)])
    dstp = jnp.concatenate([dst, jnp.full((pad_e,), NP - 1, jnp.int32)])
    sdp = jnp.stack([srcp.reshape(NW, NCHUNK, B),
                     dstp.reshape(NW, NCHUNK, B)],
                    axis=2).reshape(NW * NCHUNK, 2, B)
    eaC = jnp.concatenate([edge_attr, jnp.zeros((pad_e, D), f32)]) \
        .reshape(NW, NCHUNK, B, D).transpose(0, 1, 3, 2).reshape(-1)
    x8 = jnp.zeros((NP, 8), f32).at[:N, :D].set(x)
    zeros16 = jnp.zeros((NP, 16), f32)

    a = -0.5 / (1e-15 + sigma * sigma)
    coefs = jnp.concatenate([
        jnp.repeat(a.reshape(-1), 16),
        jnp.repeat(mu.reshape(-1), 16),
    ]).astype(f32)

    eye8 = jnp.eye(8, dtype=f32)
    g2p = jnp.zeros((16, 16), f32).at[:K * D, :].set(
        g.reshape(D, K, H).transpose(1, 0, 2).reshape(K * D, H))
    root8 = jnp.zeros((8, 16), f32).at[:D, :].set(root)
    g2bd = jnp.kron(eye8, g2p)
    rootbd = jnp.kron(eye8, root8)
    w2bd = jnp.kron(eye8, W2.astype(f32))
    w3bd = jnp.kron(eye8, W3.astype(f32))
    b1t = jnp.broadcast_to(jnp.tile(b1, 8), (8, 128)).astype(f32)
    b2t = jnp.broadcast_to(jnp.tile(b2, 8), (8, 128)).astype(f32)
    b3t = jnp.broadcast_to(jnp.tile(b3, 8), (8, 128)).astype(f32)
    ci = jnp.arange(128)
    s12 = (ci[:, None] == (ci[None, :] // 16) * 16 + 12).astype(f32)
    r8sel = (jnp.arange(512)[None, :] // G ==
             jnp.arange(8)[:, None]).astype(f32)
    wlp = jnp.zeros((16, 8), f32).at[:, :C].set(Wl)
    blp = jnp.zeros((8, 8), f32).at[:, :C].set(jnp.broadcast_to(bl, (8, C)))
    batchpk = jnp.concatenate(
        [batch, jnp.full((NP - N,), G, jnp.int32)]).reshape(NR, 8)
    xpk = x8.reshape(NR, 64)

    zpair = _sc_gmm(eaC, sdp, x8, coefs, zeros16)
    u2, dvp = _tc_a(zpair.reshape(NC, NR, 128), xpk, s12, g2bd, rootbd, b1t)
    s2 = _sc_gcn(u2.reshape(NP, 16), sdp, zeros16)
    u3 = _tc_b(s2.reshape(NC, NR, 128), u2, dvp, w2bd, b2t, relu=True)
    s3 = _sc_gcn(u3.reshape(NP, 16), sdp, zeros16)
    out8 = _tc_c(s3.reshape(NC, NR, 128), u3, dvp, w3bd, b3t, batchpk,
                 r8sel, wlp, blp)
    return out8[:, :C]


# speedup vs baseline: 40.7265x; 1.0099x over previous
import functools

import jax
import jax.numpy as jnp
from jax import lax
from jax.experimental import pallas as pl
from jax.experimental.pallas import tpu as pltpu
from jax.experimental.pallas import tpu_sc as plsc

N = 50000
E = 800000
H = 16
K = 4
D = 3
G = 64
C = 2

NC = 2
NS = 16
NW = NC * NS

NP = 51200
SLAB = NP // NS
EP = 819200
EPW = EP // NW
B = 128
NCHUNK = EPW // B
NSLABCH = SLAB // B

NB = 1024
NGRID = NP // NB

_mesh = plsc.VectorSubcoreMesh(
    core_axis_name="c", subcore_axis_name="s", num_cores=NC, num_subcores=NS
)


def _lane_const(v, dtype=jnp.float32):
    return jnp.full((16,), v, dtype)


@functools.partial(
    pl.kernel,
    out_type=jax.ShapeDtypeStruct((NC, NP, 16), jnp.float32),
    mesh=_mesh,
    compiler_params=pltpu.CompilerParams(needs_layout_passes=False, use_tc_tiling_on_sc=False),
    scratch_types=[
        pltpu.VMEM_SHARED((NP, 16), jnp.float32),
        pltpu.VMEM((8, 2, B), jnp.int32),
        pltpu.VMEM((8 * 3 * B,), jnp.float32),
        pltpu.VMEM((2, B, 8), jnp.float32),
        pltpu.VMEM((2, B, 16), jnp.float32),
        pltpu.VMEM((384,), jnp.float32),
        pltpu.SemaphoreType.DMA((8,)),
        pltpu.SemaphoreType.DMA((2,)),
        pltpu.SemaphoreType.DMA((2,)),
        pltpu.SemaphoreType.DMA((2,)),
        pltpu.SemaphoreType.DMA,
    ],
)
def _sc_gmm(eaC, sdp, x8, coefs, zeros16, out,
            zacc, sdv2, eav2, xrows2, zbuf2, cvm,
            isem, gsem, ssem, osem, zsem):
    c = lax.axis_index("c")
    s = lax.axis_index("s")
    w = c * NS + s
    r0 = s * SLAB

    pltpu.sync_copy(zeros16.at[pl.ds(0, B)], zbuf2.at[0])
    for j in range(NSLABCH):
        pltpu.async_copy(zbuf2.at[0], zacc.at[pl.ds(r0 + j * B, B)], zsem)
    pltpu.sync_copy(coefs, cvm)
    for j in range(NSLABCH):
        pltpu.make_async_copy(zbuf2.at[0], zacc.at[pl.ds(r0, B)], zsem).wait()
    for z2 in range(2):
        for j in range(B // 16):
            rows = lax.iota(jnp.int32, 16) + (j * 16)
            plsc.store_scatter(
                zbuf2, [_lane_const(z2, jnp.int32), rows,
                        _lane_const(12, jnp.int32)], _lane_const(1.0))
            for col in (13, 14, 15):
                plsc.store_scatter(
                    zbuf2, [_lane_const(z2, jnp.int32), rows,
                            _lane_const(col, jnp.int32)], _lane_const(0.0))
    plsc.subcore_barrier()

    def s_idx(i, b):
        ch = w * NCHUNK + i
        pltpu.async_copy(sdp.at[ch], sdv2.at[b], isem.at[b])
        pltpu.async_copy(eaC.at[pl.ds(ch * 3 * B, 3 * B)],
                         eav2.at[pl.ds(b * 3 * B, 3 * B)], isem.at[b])

    def w_idx(b):
        pltpu.make_async_copy(sdp.at[0], sdv2.at[b], isem.at[b]).wait()
        pltpu.make_async_copy(eaC.at[pl.ds(0, 3 * B)],
                              eav2.at[pl.ds(b * 3 * B, 3 * B)],
                              isem.at[b]).wait()

    def s_gather(i, b, g2):
        pltpu.async_copy(x8.at[sdv2.at[b, 0]], xrows2.at[g2], gsem.at[g2])

    def w_gather(g2):
        pltpu.make_async_copy(x8.at[pl.ds(0, B)], xrows2.at[g2],
                              gsem.at[g2]).wait()

    def s_scatter(b, z2):
        pltpu.async_copy(zbuf2.at[z2], zacc.at[sdv2.at[b, 1]], ssem.at[z2],
                         add=True)

    def w_scatter(z2):
        pltpu.make_async_copy(zeros16.at[pl.ds(0, B)], zbuf2.at[z2],
                              ssem.at[z2]).wait()

    av = [cvm[pl.ds(t * 16, 16)] for t in range(12)]
    mv = [cvm[pl.ds(192 + t * 16, 16)] for t in range(12)]

    def compute(b, g2, z2):
        for j in range(B // 16):
            rows = lax.iota(jnp.int32, 16) + (j * 16)
            eas = [eav2[pl.ds((b * 3 + d) * B + j * 16, 16)] for d in range(3)]
            xc = [plsc.load_gather(
                      xrows2, [_lane_const(g2, jnp.int32), rows,
                               _lane_const(d, jnp.int32)])
                  for d in range(3)]
            zsel = _lane_const(z2, jnp.int32)
            for k in range(K):
                t = None
                for d in range(3):
                    df = eas[d] - mv[k * 3 + d]
                    term = av[k * 3 + d] * df * df
                    t = term if t is None else t + term
                gk = jnp.exp(t)
                for d in range(3):
                    plsc.store_scatter(
                        zbuf2, [zsel, rows, _lane_const(k * 3 + d, jnp.int32)],
                        gk * xc[d])

    for i in range(4):
        s_idx(i, i)
    w_idx(0)
    s_gather(0, 0, 0)

    def outer(o, carry):
        for b in range(8):
            i = o * 8 + b

            @pl.when(i + 1 < NCHUNK)
            def _():
                w_idx((b + 1) % 8)
                s_gather(i + 1, (b + 1) % 8, (b + 1) % 2)

            @pl.when(i >= 2)
            def _():
                w_scatter(b % 2)

            w_gather(b % 2)
            compute(b, b % 2, b % 2)
            s_scatter(b, b % 2)

            @pl.when(i + 4 < NCHUNK)
            def _():
                s_idx(i + 4, (b + 4) % 8)
        return carry

    lax.fori_loop(0, NCHUNK // 8, outer, 0)

    for z2 in range(2):
        w_scatter(z2)
    plsc.subcore_barrier()

    for j in range(NSLABCH):
        bj = j % 2
        if j >= 2:
            pltpu.make_async_copy(zeros16.at[pl.ds(0, B)], zbuf2.at[bj],
                                  osem.at[bj]).wait()
        pltpu.sync_copy(zacc.at[pl.ds(r0 + j * B, B)], zbuf2.at[bj])
        pltpu.async_copy(zbuf2.at[bj], out.at[c, pl.ds(r0 + j * B, B)],
                         osem.at[bj])
    for bj in ((NSLABCH - 2) % 2, (NSLABCH - 1) % 2):
        pltpu.make_async_copy(zeros16.at[pl.ds(0, B)], zbuf2.at[bj],
                              osem.at[bj]).wait()


@functools.partial(
    pl.kernel,
    out_type=jax.ShapeDtypeStruct((NC, NP, 16), jnp.float32),
    mesh=_mesh,
    compiler_params=pltpu.CompilerParams(needs_layout_passes=False, use_tc_tiling_on_sc=False),
    scratch_types=[
        pltpu.VMEM_SHARED((NP, 16), jnp.float32),
        pltpu.VMEM((8, 2, B), jnp.int32),
        pltpu.VMEM((4, B, 16), jnp.float32),
        pltpu.VMEM((B, 16), jnp.float32),
        pltpu.SemaphoreType.DMA((8,)),
        pltpu.SemaphoreType.DMA((4,)),
        pltpu.SemaphoreType.DMA((4,)),
        pltpu.SemaphoreType.DMA((2,)),
        pltpu.SemaphoreType.DMA,
    ],
)
def _sc_gcn(u16, sdp, zeros16, out,
            acc, sdv2, rows2, zb, isem, gsem, ssem, osem, zsem):
    c = lax.axis_index("c")
    s = lax.axis_index("s")
    w = c * NS + s
    r0 = s * SLAB

    pltpu.sync_copy(zeros16.at[pl.ds(0, B)], zb)
    for j in range(NSLABCH):
        pltpu.async_copy(zb, acc.at[pl.ds(r0 + j * B, B)], zsem)
    for j in range(NSLABCH):
        pltpu.make_async_copy(zb, acc.at[pl.ds(r0, B)], zsem).wait()
    plsc.subcore_barrier()

    def s_idx(i, b):
        pltpu.async_copy(sdp.at[w * NCHUNK + i], sdv2.at[b], isem.at[b])

    def w_idx(b):
        pltpu.make_async_copy(sdp.at[0], sdv2.at[b], isem.at[b]).wait()

    def s_gather(b, b4):
        pltpu.async_copy(u16.at[sdv2.at[b, 0]], rows2.at[b4], gsem.at[b4])

    def w_gather(b4):
        pltpu.make_async_copy(u16.at[pl.ds(0, B)], rows2.at[b4],
                              gsem.at[b4]).wait()

    def s_scatter(b, b4):
        pltpu.async_copy(rows2.at[b4], acc.at[sdv2.at[b, 1]], ssem.at[b4],
                         add=True)

    def w_scatter(b4):
        pltpu.make_async_copy(u16.at[pl.ds(0, B)], rows2.at[b4],
                              ssem.at[b4]).wait()

    for i in range(3):
        s_idx(i, i)
    for i in range(8):
        b4 = i % 4
        if i >= 4:
            w_scatter(b4)
        w_idx(i)
        s_gather(i, b4)
        if i >= 2:
            w_gather((i - 2) % 4)
            s_scatter(i - 2, (i - 2) % 4)
        s_idx(i + 3, (i + 3) % 8)

    def outer(o, carry):
        for b in range(8):
            i = o * 8 + b
            b4 = b % 4
            w_scatter(b4)
            w_idx(b)
            s_gather(b, b4)
            pb, pb4 = (b - 2) % 8, (b - 2) % 4
            w_gather(pb4)
            s_scatter(pb, pb4)

            @pl.when(i + 3 < NCHUNK)
            def _():
                s_idx(i + 3, (b + 3) % 8)
        return carry

    lax.fori_loop(1, NCHUNK // 8, outer, 0)

    w_gather(2)
    s_scatter(6, 2)
    w_gather(3)
    s_scatter(7, 3)
    for b4 in range(4):
        w_scatter(b4)
    plsc.subcore_barrier()

    for j in range(NSLABCH):
        bj = j % 2
        if j >= 2:
            pltpu.make_async_copy(u16.at[pl.ds(0, B)], zb, osem.at[bj]).wait()
        pltpu.sync_copy(acc.at[pl.ds(r0 + j * B, B)], rows2.at[bj])
        pltpu.async_copy(rows2.at[bj], out.at[c, pl.ds(r0 + j * B, B)],
                         osem.at[bj])
    for bj in ((NSLABCH - 2) % 2, (NSLABCH - 1) % 2):
        pltpu.make_async_copy(u16.at[pl.ds(0, B)], zb, osem.at[bj]).wait()


NR = NP // 8
NBP = 640
NGRIDP = NR // NBP


def _node_mask(i):
    rowi = lax.broadcasted_iota(jnp.int32, (NBP, 128), 0)
    coli = lax.broadcasted_iota(jnp.int32, (NBP, 128), 1)
    node = (i * NBP + rowi) * 8 + (coli >> 4)
    return node < N


def _tc_a_body(zp, xp, s12, g2, r8, b1, u2o, dvo):
    Z2 = zp[0] + zp[1]
    cntb = jnp.dot(Z2, s12[...], preferred_element_type=jnp.float32)
    rcp = 1.0 / jnp.maximum(cntb, 1.0)
    agg = jnp.dot(Z2, g2[...], preferred_element_type=jnp.float32) * rcp
    xr = jnp.dot(xp[...], r8[...], preferred_element_type=jnp.float32)
    h1 = jnp.maximum(agg + xr + b1[0:1, :], 0.0)
    dinv = lax.rsqrt(cntb + 1.0)
    mask = _node_mask(pl.program_id(0))
    u2o[...] = jnp.where(mask, h1 * dinv, 0.0)
    dvo[...] = jnp.where(mask, dinv, 1.0)


def _tc_a(zpair, xpk, s12, g2bd, rootbd, b1t):
    return pl.pallas_call(
        _tc_a_body,
        grid=(NGRIDP,),
        in_specs=[
            pl.BlockSpec((NC, NBP, 128), lambda i: (0, i, 0)),
            pl.BlockSpec((NBP, 64), lambda i: (i, 0)),
            pl.BlockSpec((128, 128), lambda i: (0, 0)),
            pl.BlockSpec((128, 128), lambda i: (0, 0)),
            pl.BlockSpec((64, 128), lambda i: (0, 0)),
            pl.BlockSpec((8, 128), lambda i: (0, 0)),
        ],
        out_specs=[
            pl.BlockSpec((NBP, 128), lambda i: (i, 0)),
            pl.BlockSpec((NBP, 128), lambda i: (i, 0)),
        ],
        out_shape=[
            jax.ShapeDtypeStruct((NR, 128), jnp.float32),
            jax.ShapeDtypeStruct((NR, 128), jnp.float32),
        ],
    )(zpair, xpk, s12, g2bd, rootbd, b1t)


def _tc_b_body(sp, ub, dv, w, b, uo, *, relu):
    P = dv[...] * (sp[0] + sp[1] + ub[...])
    h = jnp.dot(P, w[...], preferred_element_type=jnp.float32) + b[0:1, :]
    if relu:
        h = jnp.maximum(h, 0.0)
    uo[...] = jnp.where(_node_mask(pl.program_id(0)), h * dv[...], 0.0)


def _tc_b(spair, u, dvp, wbd, bt, relu):
    return pl.pallas_call(
        functools.partial(_tc_b_body, relu=relu),
        grid=(NGRIDP,),
        in_specs=[
            pl.BlockSpec((NC, NBP, 128), lambda i: (0, i, 0)),
            pl.BlockSpec((NBP, 128), lambda i: (i, 0)),
            pl.BlockSpec((NBP, 128), lambda i: (i, 0)),
            pl.BlockSpec((128, 128), lambda i: (0, 0)),
            pl.BlockSpec((8, 128), lambda i: (0, 0)),
        ],
        out_specs=pl.BlockSpec((NBP, 128), lambda i: (i, 0)),
        out_shape=jax.ShapeDtypeStruct((NR, 128), jnp.float32),
    )(spair, u, dvp, wbd, bt)


def _tc_c_body(sp, ub, dv, w3, b3, bpk, r8sel, wl, bl, outo, p1, cnts):
    i = pl.program_id(0)

    @pl.when(i == 0)
    def _init():
        p1[...] = jnp.zeros((512, 128), jnp.float32)
        cnts[...] = jnp.zeros((512, 8), jnp.float32)

    P = dv[...] * (sp[0] + sp[1] + ub[...])
    h3 = jnp.dot(P, w3[...], preferred_element_type=jnp.float32) + b3[0:1, :]
    bfl = bpk[...].astype(jnp.float32)
    b512 = jnp.dot(bfl, r8sel[...], preferred_element_type=jnp.float32)
    colg = (lax.broadcasted_iota(jnp.int32, (NBP, 512), 1) %
            jnp.int32(G)).astype(jnp.float32)
    ohp = (b512 == colg).astype(jnp.float32)
    p1[...] += lax.dot_general(
        ohp, h3, (((0,), (0,)), ((), ())), preferred_element_type=jnp.float32)
    cnts[...] += lax.dot_general(
        ohp, jnp.ones((NBP, 8), jnp.float32), (((0,), (0,)), ((), ())),
        preferred_element_type=jnp.float32)

    @pl.when(i == NGRIDP - 1)
    def _fin():
        pooled = jnp.zeros((G, 16), jnp.float32)
        gc = jnp.zeros((G, 1), jnp.float32)
        for q in range(8):
            pooled = pooled + p1[q * G:(q + 1) * G, q * 16:(q + 1) * 16]
            gc = gc + cnts[q * G:(q + 1) * G, 0:1]
        pooled = pooled / jnp.maximum(gc, 1.0)
        logits = jnp.dot(pooled, wl[...],
                         preferred_element_type=jnp.float32) + bl[0:1, :]
        colmask = lax.broadcasted_iota(jnp.int32, (1, 8), 1) < C
        lm = jnp.where(colmask, logits, -1e30)
        mx = jnp.max(lm, axis=1, keepdims=True)
        ex = jnp.where(colmask, jnp.exp(lm - mx), 0.0)
        outo[...] = ex / jnp.sum(ex, axis=1, keepdims=True)


def _tc_c(spair, u3, dvp, w3bd, b3t, batchpk, r8sel, wlp, blp):
    return pl.pallas_call(
        _tc_c_body,
        grid=(NGRIDP,),
        in_specs=[
            pl.BlockSpec((NC, NBP, 128), lambda i: (0, i, 0)),
            pl.BlockSpec((NBP, 128), lambda i: (i, 0)),
            pl.BlockSpec((NBP, 128), lambda i: (i, 0)),
            pl.BlockSpec((128, 128), lambda i: (0, 0)),
            pl.BlockSpec((8, 128), lambda i: (0, 0)),
            pl.BlockSpec((NBP, 8), lambda i: (i, 0)),
            pl.BlockSpec((8, 512), lambda i: (0, 0)),
            pl.BlockSpec((16, 8), lambda i: (0, 0)),
            pl.BlockSpec((8, 8), lambda i: (0, 0)),
        ],
        out_specs=pl.BlockSpec((G, 8), lambda i: (0, 0)),
        out_shape=jax.ShapeDtypeStruct((G, 8), jnp.float32),
        scratch_shapes=[pltpu.VMEM((512, 128), jnp.float32),
                        pltpu.VMEM((512, 8), jnp.float32)],
    )(spair, u3, dvp, w3bd, b3t, batchpk, r8sel, wlp, blp)


def kernel(x, edge_index, edge_attr, batch, mu, sigma, g, root,
           b1, W2, b2, W3, b3, Wl, bl):
    f32 = jnp.float32
    src = edge_index[0]
    dst = edge_index[1]

    pad_e = EP - E
    srcp = jnp.concatenate([src, jnp.full((pad_e,), NP - 1, jnp.int32)])
    dstp = jnp.concatenate([dst, jnp.full((pad_e,), NP - 1, jnp.int32)])
    sdp = jnp.stack([srcp.reshape(NW, NCHUNK, B),
                     dstp.reshape(NW, NCHUNK, B)],
                    axis=2).reshape(NW * NCHUNK, 2, B)
    eaC = jnp.concatenate([edge_attr, jnp.zeros((pad_e, D), f32)]) \
        .reshape(NW, NCHUNK, B, D).transpose(0, 1, 3, 2).reshape(-1)
    x8 = jnp.zeros((NP, 8), f32).at[:N, :D].set(x)
    zeros16 = jnp.zeros((NP, 16), f32)

    a = -0.5 / (1e-15 + sigma * sigma)
    coefs = jnp.concatenate([
        jnp.repeat(a.reshape(-1), 16),
        jnp.repeat(mu.reshape(-1), 16),
    ]).astype(f32)

    eye8 = jnp.eye(8, dtype=f32)
    g2p = jnp.zeros((16, 16), f32).at[:K * D, :].set(
        g.reshape(D, K, H).transpose(1, 0, 2).reshape(K * D, H))
    root8 = jnp.zeros((8, 16), f32).at[:D, :].set(root)
    g2bd = jnp.kron(eye8, g2p)
    rootbd = jnp.kron(eye8, root8)
    w2bd = jnp.kron(eye8, W2.astype(f32))
    w3bd = jnp.kron(eye8, W3.astype(f32))
    b1t = jnp.broadcast_to(jnp.tile(b1, 8), (8, 128)).astype(f32)
    b2t = jnp.broadcast_to(jnp.tile(b2, 8), (8, 128)).astype(f32)
    b3t = jnp.broadcast_to(jnp.tile(b3, 8), (8, 128)).astype(f32)
    ci = jnp.arange(128)
    s12 = (ci[:, None] == (ci[None, :] // 16) * 16 + 12).astype(f32)
    r8sel = (jnp.arange(512)[None, :] // G ==
             jnp.arange(8)[:, None]).astype(f32)
    wlp = jnp.zeros((16, 8), f32).at[:, :C].set(Wl)
    blp = jnp.zeros((8, 8), f32).at[:, :C].set(jnp.broadcast_to(bl, (8, C)))
    batchpk = jnp.concatenate(
        [batch, jnp.full((NP - N,), G, jnp.int32)]).reshape(NR, 8)
    xpk = x8.reshape(NR, 64)

    zpair = _sc_gmm(eaC, sdp, x8, coefs, zeros16)
    u2, dvp = _tc_a(zpair.reshape(NC, NR, 128), xpk, s12, g2bd, rootbd, b1t)
    s2 = _sc_gcn(u2.reshape(NP, 16), sdp, zeros16)
    u3 = _tc_b(s2.reshape(NC, NR, 128), u2, dvp, w2bd, b2t, relu=True)
    s3 = _sc_gcn(u3.reshape(NP, 16), sdp, zeros16)
    out8 = _tc_c(s3.reshape(NC, NR, 128), u3, dvp, w3bd, b3t, batchpk,
                 r8sel, wlp, blp)
    return out8[:, :C]
